# Initial kernel scaffold; baseline (speedup 1.0000x reference)
import jax
import jax.numpy as jnp
from jax.experimental import pallas as pl


def kernel(h, m_ids, m_gs, pos, params):
    raise NotImplementedError("write your pallas kernel here")


# speedup vs baseline: 2.7338x; 2.7338x over previous
import functools

import jax
import jax.numpy as jnp
from jax import lax
from jax.experimental import pallas as pl
from jax.experimental.pallas import tpu as pltpu
from jax.experimental.pallas import tpu_sc as plsc

F32 = jnp.float32
I32 = jnp.int32
NC, NS = 2, 16
NW = NC * NS
CH = 80
LAT = 128


def _wid():
    return lax.axis_index("s") * NC + lax.axis_index("c")


def _mesh():
    return plsc.VectorSubcoreMesh(core_axis_name="c", subcore_axis_name="s")


def _sc_gather(tables, idxs):
    P = len(tables)
    B = idxs[0].shape[0]
    span = B // NW
    assert B % NW == 0 and span % CH == 0
    nch = span // CH
    out_type = tuple(jax.ShapeDtypeStruct((B, t.shape[1]), F32) for t in tables)
    scratch = []
    for t in tables:
        scratch += [pltpu.VMEM((CH,), I32), pltpu.VMEM((CH, t.shape[1]), F32)]
    scratch.append(pltpu.SemaphoreType.DMA)

    @functools.partial(pl.kernel, out_type=out_type, mesh=_mesh(),
                       scratch_types=scratch)
    def k(*refs):
        t_refs = refs[:P]
        i_refs = refs[P:2 * P]
        o_refs = refs[2 * P:3 * P]
        sc = refs[3 * P:]
        sem = sc[-1]
        base = _wid() * span

        def body(c, carry):
            off = base + c * CH
            for p in range(P):
                iv, rv = sc[2 * p], sc[2 * p + 1]
                pltpu.sync_copy(i_refs[p].at[pl.ds(off, CH)], iv)
                pltpu.async_copy(t_refs[p].at[iv], rv, sem).wait()
                pltpu.sync_copy(rv, o_refs[p].at[pl.ds(off, CH)])
            return carry

        lax.fori_loop(0, nch, body, 0)

    return k(*tables, *idxs)


def _scatter_epilogue(sh, o_refs, cid, sid, rpt, rlast, P):
    @pl.when(sid < NS - 1)
    def _():
        for p in range(P):
            pltpu.sync_copy(sh[p].at[pl.ds(sid * rpt, rpt)],
                            o_refs[p].at[cid, pl.ds(sid * rpt, rpt)])

    @pl.when(sid == NS - 1)
    def _():
        for p in range(P):
            pltpu.sync_copy(sh[p].at[pl.ds((NS - 1) * rpt, rlast)],
                            o_refs[p].at[cid, pl.ds((NS - 1) * rpt, rlast)])


def _sc_scatter(vals, idx, n, zeros, ones_width=None):
    synth = ones_width is not None
    P = 1 if synth else len(vals)
    widths = [ones_width] if synth else [v.shape[1] for v in vals]
    e = idx.shape[0]
    spc = e // NC
    sps = spc // NS
    assert e % (NC * NS) == 0 and sps % CH == 0
    nch = sps // CH
    rpt = ((n // NS) // 8) * 8
    rlast = n - (NS - 1) * rpt
    out_type = tuple(jax.ShapeDtypeStruct((NC, n, d), F32) for d in widths)
    scratch = [pltpu.VMEM((CH,), I32)]
    for d in widths:
        scratch.append(pltpu.VMEM((CH, d), F32))
    for d in widths:
        scratch.append(pltpu.VMEM_SHARED((n, d), F32))
    n_in = 0 if synth else P

    @functools.partial(pl.kernel, out_type=out_type, mesh=_mesh(),
                       scratch_types=scratch)
    def k(*refs):
        v_refs = refs[:n_in]
        i_ref = refs[n_in]
        z_refs = refs[n_in + 1:n_in + 1 + P]
        o_refs = refs[n_in + 1 + P:n_in + 1 + 2 * P]
        ivb = refs[n_in + 1 + 2 * P]
        vb = refs[n_in + 2 + 2 * P:n_in + 2 + 3 * P]
        sh = refs[n_in + 2 + 3 * P:n_in + 2 + 4 * P]
        cid = lax.axis_index("c")
        sid = lax.axis_index("s")

        @pl.when(sid == 0)
        def _():
            for p in range(P):
                pltpu.sync_copy(z_refs[p], sh[p])

        if synth:
            def fill(r, carry):
                for s in range(widths[0] // 16):
                    vb[0][r, pl.ds(s * 16, 16)] = jnp.ones((16,), F32)
                return carry

            lax.fori_loop(0, CH, fill, 0)

        plsc.subcore_barrier()
        base = cid * spc + sid * sps

        def body(c, carry):
            off = base + c * CH
            pltpu.sync_copy(i_ref.at[pl.ds(off, CH)], ivb)
            for p in range(P):
                if not synth:
                    pltpu.sync_copy(v_refs[p].at[pl.ds(off, CH)], vb[p])
                pltpu.sync_copy(vb[p], sh[p].at[ivb], add=True)
            return carry

        lax.fori_loop(0, nch, body, 0)
        plsc.subcore_barrier()
        _scatter_epilogue(sh, o_refs, cid, sid, rpt, rlast, P)

    if synth:
        return k(idx, *zeros)
    return k(*vals, idx, *zeros)


def _sc_gather_scatter(table, gidx, sidx, n, zeros):
    d = table.shape[1]
    e = gidx.shape[0]
    spc = e // NC
    sps = spc // NS
    assert e % (NC * NS) == 0 and sps % CH == 0
    nch = sps // CH
    rpt = ((n // NS) // 8) * 8
    rlast = n - (NS - 1) * rpt

    @functools.partial(
        pl.kernel, out_type=(jax.ShapeDtypeStruct((NC, n, d), F32),),
        mesh=_mesh(),
        scratch_types=[pltpu.VMEM((CH,), I32), pltpu.VMEM((CH,), I32),
                       pltpu.VMEM((CH, d), F32), pltpu.VMEM_SHARED((n, d), F32),
                       pltpu.SemaphoreType.DMA])
    def k(t_ref, g_ref, s_ref, z_ref, o_ref, gvb, svb, vb, sh, sem):
        cid = lax.axis_index("c")
        sid = lax.axis_index("s")

        @pl.when(sid == 0)
        def _():
            pltpu.sync_copy(z_ref, sh)

        plsc.subcore_barrier()
        base = cid * spc + sid * sps

        def body(c, carry):
            off = base + c * CH
            pltpu.sync_copy(g_ref.at[pl.ds(off, CH)], gvb)
            pltpu.async_copy(t_ref.at[gvb], vb, sem).wait()
            pltpu.sync_copy(s_ref.at[pl.ds(off, CH)], svb)
            pltpu.sync_copy(vb, sh.at[svb], add=True)
            return carry

        lax.fori_loop(0, nch, body, 0)
        plsc.subcore_barrier()
        _scatter_epilogue([sh], [o_ref], cid, sid, rpt, rlast, 1)

    return k(table, gidx, sidx, zeros)[0]


def _sc_win(idp, n, padn):
    np2 = ((n + 127) // 128) * 128
    spw = padn // NW
    ng = spw // 16

    @functools.partial(
        pl.kernel, out_type=jax.ShapeDtypeStruct((NW * np2,), I32),
        mesh=_mesh(),
        scratch_types=[pltpu.VMEM((padn,), I32), pltpu.VMEM((np2 + 16,), I32)])
    def k(id_ref, o_ref, idv, acc):
        w = _wid()
        pltpu.sync_copy(id_ref, idv)

        def z(c, carry):
            acc[pl.ds(c * 16, 16)] = jnp.full((16,), -1, I32)
            return carry

        lax.fori_loop(0, (np2 + 16) // 16, z, 0)
        k0 = w * spw
        iot = lax.iota(I32, 16)

        def outer(g, carry):
            kbase = k0 + g * 16
            idvec = idv[pl.ds(kbase, 16)]
            for l in range(16):
                kk = kbase + l

                @pl.when(kk < n)
                def _():
                    t = idvec[l]
                    cur = acc[pl.ds(t, 16)]
                    acc[pl.ds(t, 16)] = jnp.where(iot == 0, kk, cur)
            return carry

        lax.fori_loop(0, ng, outer, 0)
        pltpu.sync_copy(acc.at[pl.ds(0, np2)], o_ref.at[pl.ds(w * np2, np2)])

    return k(idp).reshape(NW, np2)


_BN = 1000


def _row_spec(d):
    return pl.BlockSpec((_BN, d), lambda i: (i, 0))


def _full_spec(r, c):
    return pl.BlockSpec((r, c), lambda i: (0, 0))


def _tc_prep(x_list, posp, wa, wb, c3):
    n, lat = x_list[0].shape
    nx = len(x_list)

    def body(*refs):
        xs = refs[:nx]
        pp, wa_r, wb_r, c3_r, a_r, b_r = refs[nx:]
        x = xs[0][...]
        for r in xs[1:]:
            x = x + r[...]
        pc = jnp.dot(pp[...], c3_r[...], preferred_element_type=F32)
        a_r[...] = jnp.dot(x, wa_r[...], preferred_element_type=F32) + pc
        b_r[...] = jnp.dot(x, wb_r[...], preferred_element_type=F32) - pc

    return pl.pallas_call(
        body, grid=(n // _BN,),
        in_specs=[_row_spec(lat)] * nx + [_row_spec(LAT), _full_spec(lat, lat),
                                          _full_spec(lat, lat),
                                          _full_spec(LAT, lat)],
        out_specs=[_row_spec(lat)] * 2,
        out_shape=[jax.ShapeDtypeStruct((n, lat), F32)] * 2,
    )(*x_list, posp, wa, wb, c3)


def _ln(y, g, b):
    mu = jnp.mean(y, axis=-1, keepdims=True)
    yc = y - mu
    var = jnp.mean(yc * yc, axis=-1, keepdims=True)
    return yc * lax.rsqrt(var + 1e-5) * g + b


def _tc_edge(ga, gb, pi, pj, c4, b0, w1, b1, g, b):
    e, lat = ga.shape

    def body(ga_r, gb_r, pi_r, pj_r, c4_r, b0_r, w1_r, b1_r, g_r, b_r, o_r):
        d = pi_r[...] - pj_r[...]
        nrm = jnp.sqrt(jnp.sum(d * d, axis=-1, keepdims=True) + 1e-12)
        y = ga_r[...] + gb_r[...] + nrm * c4_r[...] + b0_r[...]
        y = jnp.maximum(y, 0.0)
        y = jnp.dot(y, w1_r[...], preferred_element_type=F32) + b1_r[...]
        o_r[...] = _ln(y, g_r[...], b_r[...])

    return pl.pallas_call(
        body, grid=(e // _BN,),
        in_specs=[_row_spec(lat)] * 4
        + [_full_spec(1, lat), _full_spec(1, lat), _full_spec(lat, lat),
           _full_spec(1, lat), _full_spec(1, lat), _full_spec(1, lat)],
        out_specs=_row_spec(lat),
        out_shape=jax.ShapeDtypeStruct((e, lat), F32))(
            ga, gb, pi, pj, c4, b0, w1, b1, g, b)


def _tc_node(x_list, a0, a1, wx, wa, b0, w1, b1, g, b, extra=None):
    n, lat = x_list[0].shape
    nx = len(x_list)
    ins = list(x_list) + [a0, a1, wx, wa, b0, w1, b1, g, b]
    if extra is not None:
        ins.append(extra)
    ne = extra is not None

    def body(*refs):
        xs = refs[:nx]
        a0_r, a1_r, wx_r, wa_r, b0_r, w1_r, b1_r, g_r, b_r = refs[nx:nx + 9]
        o_r = refs[-1]
        x = xs[0][...]
        for r in xs[1:]:
            x = x + r[...]
        ag = a0_r[...] + a1_r[...]
        y = (jnp.dot(x, wx_r[...], preferred_element_type=F32)
             + jnp.dot(ag, wa_r[...], preferred_element_type=F32) + b0_r[...])
        y = jnp.maximum(y, 0.0)
        y = jnp.dot(y, w1_r[...], preferred_element_type=F32) + b1_r[...]
        y = _ln(y, g_r[...], b_r[...]) + x
        if ne:
            y = y + refs[nx + 9][...]
        o_r[...] = y

    return pl.pallas_call(
        body, grid=(n // _BN,),
        in_specs=[_row_spec(lat)] * nx
        + [_row_spec(lat), _row_spec(lat), _full_spec(lat, lat),
           _full_spec(lat, lat), _full_spec(1, lat), _full_spec(lat, lat),
           _full_spec(1, lat), _full_spec(1, lat), _full_spec(1, lat)]
        + ([_row_spec(lat)] if ne else []),
        out_specs=_row_spec(lat),
        out_shape=jax.ShapeDtypeStruct((n, lat), F32))(*ins)


def _tc_combine(parts, mults, post=None):
    nc, n, d = parts.shape

    def body(*refs):
        in_r = refs[0]
        o_r = refs[-1]
        s = in_r[0] + in_r[1]
        if post == "inv":
            s = 1.0 / s
        elif post == "inveps":
            s = 1.0 / (s + 1e-12)
        for m_r in refs[1:-1]:
            s = s * m_r[...]
        o_r[...] = s

    return pl.pallas_call(
        body, grid=(n // _BN,),
        in_specs=[pl.BlockSpec((nc, _BN, d), lambda i: (0, i, 0))]
        + [_row_spec(d)] * len(mults),
        out_specs=_row_spec(d),
        out_shape=jax.ShapeDtypeStruct((n, d), F32))(parts, *mults)


def _tc_mul(a, b, col=None):
    n, d = a.shape
    ins = [a, b] + ([col] if col is not None else [])

    def body(*refs):
        o_r = refs[-1]
        y = refs[0][...] * refs[1][...]
        if col is not None:
            y = y * refs[2][...]
        o_r[...] = y

    return pl.pallas_call(
        body, grid=(n // _BN,),
        in_specs=[_row_spec(d), _row_spec(d)]
        + ([_row_spec(1)] if col is not None else []),
        out_specs=_row_spec(d),
        out_shape=jax.ShapeDtypeStruct((n, d), F32))(*ins)


def _tc_winmax(parts):
    p, n = parts.shape

    def body(in_r, w_r, m_r):
        m = jnp.max(in_r[...], axis=0, keepdims=True)
        w_r[...] = jnp.maximum(m, 0)
        m_r[...] = jnp.where(m >= 0, 1.0, 0.0).astype(F32)

    return pl.pallas_call(
        body, grid=(1,),
        in_specs=[pl.BlockSpec((p, n), lambda i: (0, 0))],
        out_specs=[pl.BlockSpec((1, n), lambda i: (0, 0))] * 2,
        out_shape=[jax.ShapeDtypeStruct((1, n), I32),
                   jax.ShapeDtypeStruct((1, n), F32)])(parts)


def _edge_w(p, lat):
    w0 = p["w0"]
    c3 = jnp.pad(w0[0:3], ((0, LAT - 3), (0, 0)))
    c4 = w0[3:4]
    wa = w0[4:4 + lat]
    wb = w0[4 + lat:4 + 2 * lat]
    return (c3, c4, wa, wb, p["b0"].reshape(1, -1), p["w1"],
            p["b1"].reshape(1, -1), p["ln_g"].reshape(1, -1),
            p["ln_b"].reshape(1, -1))


def _node_w(p, lat):
    return (p["w0"][:lat], p["w0"][lat:], p["b0"].reshape(1, -1), p["w1"],
            p["b1"].reshape(1, -1), p["ln_g"].reshape(1, -1),
            p["ln_b"].reshape(1, -1))


def _gmp(x_list, pi, pj, i_idx, j_idx, posk, pe, pn, n, lat, zz, extra=None):
    c3, c4, wa, wb, b0e, w1e, b1e, ge, be = _edge_w(pe, lat)
    a_t, b_t = _tc_prep(x_list, posk, wa, wb, c3)
    ga, gb = _sc_gather([a_t, b_t], [i_idx, j_idx])
    emb = _tc_edge(ga, gb, pi, pj, c4, b0e, w1e, b1e, ge, be)
    (agp,) = _sc_scatter([emb], j_idx, n, [zz])
    wx, wac, b0n, w1n, b1n, gn, bn = _node_w(pn, lat)
    return _tc_node(x_list, agp[0], agp[1], wx, wac, b0n, w1n, b1n, gn, bn,
                    extra=extra)


def kernel(h, m_ids, m_gs, pos, params):
    n, lat = h.shape
    e = m_gs.shape[-1]
    i0 = m_gs[0, 0]
    j0 = m_gs[0, 1]
    i1 = m_gs[1, 0]
    j1 = m_gs[1, 1]
    ids0 = m_ids[0]
    padn = ((n + NW * CH - 1) // (NW * CH)) * NW * CH

    posp = jnp.pad(pos, ((0, 0), (0, LAT - pos.shape[1])))
    zz128 = jnp.zeros((n, lat), F32)

    pi0, pj0 = _sc_gather([posp, posp], [i0, j0])
    h1 = _gmp([h], pi0, pj0, i0, j0, posp, params["down0"]["mlp_edge"],
              params["down0"]["mlp_node"], n, lat, zz128)

    (degp,) = _sc_scatter(None, i0, n, [zz128], ones_width=lat)
    nw128 = _tc_combine(degp, [], post="inv")
    awp = _sc_gather_scatter(nw128, i0, j0, n, zz128)
    iaw128 = _tc_combine(awp, [], post="inveps")

    hw1 = _tc_mul(h1, nw128)
    posw = _tc_mul(posp, nw128)
    shp = _sc_gather_scatter(hw1, i0, j0, n, zz128)
    spp = _sc_gather_scatter(posw, i0, j0, n, zz128)
    h2f = _tc_combine(shp, [iaw128])
    p2f = _tc_combine(spp, [iaw128])
    idp = jnp.pad(ids0, (0, padn - n))
    h2p, p2p = _sc_gather([h2f, p2f], [idp, idp])
    h2 = h2p[:n]
    p2 = p2p[:n]

    pi1, pj1 = _sc_gather([p2, p2], [i1, j1])
    h3 = _gmp([h2], pi1, pj1, i1, j1, p2, params["bottom"]["mlp_edge"],
              params["bottom"]["mlp_node"], n, lat, zz128)

    winp = _sc_win(idp, n, padn)
    winc, maskf = _tc_winmax(winp)
    wpad = jnp.pad(winc.reshape(-1)[:n], (0, padn - n))
    (h4p,) = _sc_gather([h3], [wpad])

    hw4 = _tc_mul(h4p[:n], iaw128, col=maskf.reshape(-1)[:n].reshape(n, 1))
    up_ = _sc_gather_scatter(hw4, j0, i0, n, zz128)
    hu = _tc_combine(up_, [nw128])

    out = _gmp([hu], pi0, pj0, i0, j0, posp, params["up0"]["mlp_edge"],
               params["up0"]["mlp_node"], n, lat, zz128, extra=h1)
    return out


# speedup vs baseline: 3.5961x; 1.3154x over previous
import functools

import jax
import jax.numpy as jnp
from jax import lax
from jax.experimental import pallas as pl
from jax.experimental.pallas import tpu as pltpu
from jax.experimental.pallas import tpu_sc as plsc

F32 = jnp.float32
I32 = jnp.int32
NC, NS = 2, 16
NW = NC * NS
CH = 40
LAT = 128


def _wid():
    return lax.axis_index("s") * NC + lax.axis_index("c")


def _mesh():
    return plsc.VectorSubcoreMesh(core_axis_name="c", subcore_axis_name="s")


def _pipe_depth(nch, p):
    for kd in (5, 4, 2):
        if nch % kd == 0 and kd * p <= 10:
            return kd
    return 1


def _sc_gather(tables, idxs):
    P = len(tables)
    B = idxs[0].shape[0]
    span = B // NW
    assert B % NW == 0 and span % CH == 0
    nch = span // CH
    K = _pipe_depth(nch, P)
    ng = nch // K
    out_type = tuple(jax.ShapeDtypeStruct((B, t.shape[1]), F32) for t in tables)
    scratch = []
    for t in tables:
        scratch += [pltpu.VMEM((K, CH), I32), pltpu.VMEM((K, CH, t.shape[1]), F32)]
    scratch += [pltpu.SemaphoreType.DMA, pltpu.SemaphoreType.DMA]

    @functools.partial(pl.kernel, out_type=out_type, mesh=_mesh(),
                       scratch_types=scratch)
    def k(*refs):
        t_refs = refs[:P]
        i_refs = refs[P:2 * P]
        o_refs = refs[2 * P:3 * P]
        sc = refs[3 * P:]
        gsem, wsem = sc[-2], sc[-1]
        base = _wid() * span

        def body(g, carry):
            for u in range(K):
                off = base + (g * K + u) * CH
                for p in range(P):
                    iv = sc[2 * p].at[u]
                    rv = sc[2 * p + 1].at[u]
                    dst = o_refs[p].at[pl.ds(off, CH)]

                    @pl.when(g > 0)
                    def _():
                        pltpu.make_async_copy(rv, dst, wsem).wait()

                    pltpu.sync_copy(i_refs[p].at[pl.ds(off, CH)], iv)
                    pltpu.async_copy(t_refs[p].at[iv], rv, gsem)
            for u in range(K):
                off = base + (g * K + u) * CH
                for p in range(P):
                    iv = sc[2 * p].at[u]
                    rv = sc[2 * p + 1].at[u]
                    pltpu.make_async_copy(t_refs[p].at[iv], rv, gsem).wait()
                    pltpu.async_copy(rv, o_refs[p].at[pl.ds(off, CH)], wsem)
            return carry

        lax.fori_loop(0, ng, body, 0)
        for u in range(K):
            off = base + (nch - K + u) * CH
            for p in range(P):
                rv = sc[2 * p + 1].at[u]
                pltpu.make_async_copy(rv, o_refs[p].at[pl.ds(off, CH)],
                                      wsem).wait()

    return k(*tables, *idxs)


def _scatter_epilogue(sh, o_refs, cid, sid, rpt, rlast, P):
    @pl.when(sid < NS - 1)
    def _():
        for p in range(P):
            pltpu.sync_copy(sh[p].at[pl.ds(sid * rpt, rpt)],
                            o_refs[p].at[cid, pl.ds(sid * rpt, rpt)])

    @pl.when(sid == NS - 1)
    def _():
        for p in range(P):
            pltpu.sync_copy(sh[p].at[pl.ds((NS - 1) * rpt, rlast)],
                            o_refs[p].at[cid, pl.ds((NS - 1) * rpt, rlast)])


def _sc_scatter(vals, idx, n, zeros, ones_width=None):
    synth = ones_width is not None
    P = 1 if synth else len(vals)
    widths = [ones_width] if synth else [v.shape[1] for v in vals]
    e = idx.shape[0]
    spc = e // NC
    sps = spc // NS
    assert e % (NC * NS) == 0 and sps % CH == 0
    nch = sps // CH
    rpt = ((n // NS) // 8) * 8
    rlast = n - (NS - 1) * rpt
    out_type = tuple(jax.ShapeDtypeStruct((NC, n, d), F32) for d in widths)
    n_in = 0 if synth else P
    K = _pipe_depth(nch, P)
    ng = nch // K
    scratch = [pltpu.VMEM((K, CH), I32)]
    for d in widths:
        scratch.append(pltpu.VMEM((1 if synth else K, CH, d), F32))
    for d in widths:
        scratch.append(pltpu.VMEM_SHARED((n, d), F32))
    scratch += [pltpu.SemaphoreType.DMA, pltpu.SemaphoreType.DMA]

    @functools.partial(pl.kernel, out_type=out_type, mesh=_mesh(),
                       scratch_types=scratch)
    def k(*refs):
        v_refs = refs[:n_in]
        i_ref = refs[n_in]
        z_refs = refs[n_in + 1:n_in + 1 + P]
        o_refs = refs[n_in + 1 + P:n_in + 1 + 2 * P]
        ivb = refs[n_in + 1 + 2 * P]
        vb = refs[n_in + 2 + 2 * P:n_in + 2 + 3 * P]
        sh = refs[n_in + 2 + 3 * P:n_in + 2 + 4 * P]
        vsem, wsem = refs[-2], refs[-1]
        cid = lax.axis_index("c")
        sid = lax.axis_index("s")

        @pl.when(sid == 0)
        def _():
            for p in range(P):
                pltpu.sync_copy(z_refs[p], sh[p])

        if synth:
            def fill(r, carry):
                for s in range(widths[0] // 16):
                    vb[0][0, r, pl.ds(s * 16, 16)] = jnp.ones((16,), F32)
                return carry

            lax.fori_loop(0, CH, fill, 0)

        plsc.subcore_barrier()
        base = cid * spc + sid * sps

        def body(g, carry):
            for u in range(K):
                off = base + (g * K + u) * CH
                iv = ivb.at[u]
                for p in range(P):
                    bv = vb[p].at[0 if synth else u]

                    @pl.when(g > 0)
                    def _():
                        pltpu.make_async_copy(bv, sh[p].at[iv], wsem).wait()

                pltpu.sync_copy(i_ref.at[pl.ds(off, CH)], iv)
                if not synth:
                    for p in range(P):
                        pltpu.async_copy(v_refs[p].at[pl.ds(off, CH)],
                                         vb[p].at[u], vsem)
            for u in range(K):
                off = base + (g * K + u) * CH
                iv = ivb.at[u]
                for p in range(P):
                    bv = vb[p].at[0 if synth else u]
                    if not synth:
                        pltpu.make_async_copy(v_refs[p].at[pl.ds(off, CH)],
                                              bv, vsem).wait()
                    pltpu.async_copy(bv, sh[p].at[iv], wsem, add=True)
            return carry

        lax.fori_loop(0, ng, body, 0)
        for u in range(K):
            iv = ivb.at[u]
            for p in range(P):
                bv = vb[p].at[0 if synth else u]
                pltpu.make_async_copy(bv, sh[p].at[iv], wsem).wait()
        plsc.subcore_barrier()
        _scatter_epilogue(sh, o_refs, cid, sid, rpt, rlast, P)

    if synth:
        return k(idx, *zeros)
    return k(*vals, idx, *zeros)


def _sc_gather_scatter(table, gidx, sidx, n, zeros):
    d = table.shape[1]
    e = gidx.shape[0]
    spc = e // NC
    sps = spc // NS
    assert e % (NC * NS) == 0 and sps % CH == 0
    nch = sps // CH
    rpt = ((n // NS) // 8) * 8
    rlast = n - (NS - 1) * rpt

    K = _pipe_depth(nch, 1)
    ng = nch // K

    @functools.partial(
        pl.kernel, out_type=(jax.ShapeDtypeStruct((NC, n, d), F32),),
        mesh=_mesh(),
        scratch_types=[pltpu.VMEM((K, CH), I32), pltpu.VMEM((K, CH), I32),
                       pltpu.VMEM((K, CH, d), F32),
                       pltpu.VMEM_SHARED((n, d), F32),
                       pltpu.SemaphoreType.DMA, pltpu.SemaphoreType.DMA])
    def k(t_ref, g_ref, s_ref, z_ref, o_ref, gvb, svb, vb, sh, gsem, wsem):
        cid = lax.axis_index("c")
        sid = lax.axis_index("s")

        @pl.when(sid == 0)
        def _():
            pltpu.sync_copy(z_ref, sh)

        plsc.subcore_barrier()
        base = cid * spc + sid * sps

        def body(g, carry):
            for u in range(K):
                off = base + (g * K + u) * CH
                gv, sv, bv = gvb.at[u], svb.at[u], vb.at[u]

                @pl.when(g > 0)
                def _():
                    pltpu.make_async_copy(bv, sh.at[sv], wsem).wait()

                pltpu.sync_copy(g_ref.at[pl.ds(off, CH)], gv)
                pltpu.sync_copy(s_ref.at[pl.ds(off, CH)], sv)
                pltpu.async_copy(t_ref.at[gv], bv, gsem)
            for u in range(K):
                gv, sv, bv = gvb.at[u], svb.at[u], vb.at[u]
                pltpu.make_async_copy(t_ref.at[gv], bv, gsem).wait()
                pltpu.async_copy(bv, sh.at[sv], wsem, add=True)
            return carry

        lax.fori_loop(0, ng, body, 0)
        for u in range(K):
            pltpu.make_async_copy(vb.at[u], sh.at[svb.at[u]], wsem).wait()
        plsc.subcore_barrier()
        _scatter_epilogue([sh], [o_ref], cid, sid, rpt, rlast, 1)

    return k(table, gidx, sidx, zeros)[0]


def _sc_win(idp, n, padn):
    np2 = ((n + 127) // 128) * 128
    spw = padn // NW
    ng = spw // 16

    @functools.partial(
        pl.kernel, out_type=jax.ShapeDtypeStruct((NW * np2,), I32),
        mesh=_mesh(),
        scratch_types=[pltpu.VMEM((padn,), I32), pltpu.VMEM((np2 + 16,), I32)])
    def k(id_ref, o_ref, idv, acc):
        w = _wid()
        pltpu.sync_copy(id_ref, idv)

        def z(c, carry):
            acc[pl.ds(c * 16, 16)] = jnp.full((16,), -1, I32)
            return carry

        lax.fori_loop(0, (np2 + 16) // 16, z, 0)
        k0 = w * spw
        iot = lax.iota(I32, 16)

        def outer(g, carry):
            kbase = k0 + g * 16
            idvec = idv[pl.ds(kbase, 16)]
            for l in range(16):
                kk = kbase + l

                @pl.when(kk < n)
                def _():
                    t = idvec[l]
                    cur = acc[pl.ds(t, 16)]
                    acc[pl.ds(t, 16)] = jnp.where(iot == 0, kk, cur)
            return carry

        lax.fori_loop(0, ng, outer, 0)
        pltpu.sync_copy(acc.at[pl.ds(0, np2)], o_ref.at[pl.ds(w * np2, np2)])

    return k(idp).reshape(NW, np2)


_BN = 1000


def _row_spec(d):
    return pl.BlockSpec((_BN, d), lambda i: (i, 0))


def _full_spec(r, c):
    return pl.BlockSpec((r, c), lambda i: (0, 0))


def _tc_prep(x_list, posp, wa, wb, c3):
    n, lat = x_list[0].shape
    nx = len(x_list)

    def body(*refs):
        xs = refs[:nx]
        pp, wa_r, wb_r, c3_r, a_r, b_r = refs[nx:]
        x = xs[0][...]
        for r in xs[1:]:
            x = x + r[...]
        pc = jnp.dot(pp[...], c3_r[...], preferred_element_type=F32)
        a_r[...] = jnp.dot(x, wa_r[...], preferred_element_type=F32) + pc
        b_r[...] = jnp.dot(x, wb_r[...], preferred_element_type=F32) - pc

    return pl.pallas_call(
        body, grid=(n // _BN,),
        in_specs=[_row_spec(lat)] * nx + [_row_spec(LAT), _full_spec(lat, lat),
                                          _full_spec(lat, lat),
                                          _full_spec(LAT, lat)],
        out_specs=[_row_spec(lat)] * 2,
        out_shape=[jax.ShapeDtypeStruct((n, lat), F32)] * 2,
    )(*x_list, posp, wa, wb, c3)


def _ln(y, g, b):
    mu = jnp.mean(y, axis=-1, keepdims=True)
    yc = y - mu
    var = jnp.mean(yc * yc, axis=-1, keepdims=True)
    return yc * lax.rsqrt(var + 1e-5) * g + b


def _tc_edge(ga, gb, pi, pj, c4, b0, w1, b1, g, b):
    e, lat = ga.shape

    def body(ga_r, gb_r, pi_r, pj_r, c4_r, b0_r, w1_r, b1_r, g_r, b_r, o_r):
        d = pi_r[...] - pj_r[...]
        nrm = jnp.sqrt(jnp.sum(d * d, axis=-1, keepdims=True) + 1e-12)
        y = ga_r[...] + gb_r[...] + nrm * c4_r[...] + b0_r[...]
        y = jnp.maximum(y, 0.0)
        y = jnp.dot(y, w1_r[...], preferred_element_type=F32) + b1_r[...]
        o_r[...] = _ln(y, g_r[...], b_r[...])

    return pl.pallas_call(
        body, grid=(e // _BN,),
        in_specs=[_row_spec(lat)] * 4
        + [_full_spec(1, lat), _full_spec(1, lat), _full_spec(lat, lat),
           _full_spec(1, lat), _full_spec(1, lat), _full_spec(1, lat)],
        out_specs=_row_spec(lat),
        out_shape=jax.ShapeDtypeStruct((e, lat), F32))(
            ga, gb, pi, pj, c4, b0, w1, b1, g, b)


def _tc_node(x_list, a0, a1, wx, wa, b0, w1, b1, g, b, extra=None):
    n, lat = x_list[0].shape
    nx = len(x_list)
    ins = list(x_list) + [a0, a1, wx, wa, b0, w1, b1, g, b]
    if extra is not None:
        ins.append(extra)
    ne = extra is not None

    def body(*refs):
        xs = refs[:nx]
        a0_r, a1_r, wx_r, wa_r, b0_r, w1_r, b1_r, g_r, b_r = refs[nx:nx + 9]
        o_r = refs[-1]
        x = xs[0][...]
        for r in xs[1:]:
            x = x + r[...]
        ag = a0_r[...] + a1_r[...]
        y = (jnp.dot(x, wx_r[...], preferred_element_type=F32)
             + jnp.dot(ag, wa_r[...], preferred_element_type=F32) + b0_r[...])
        y = jnp.maximum(y, 0.0)
        y = jnp.dot(y, w1_r[...], preferred_element_type=F32) + b1_r[...]
        y = _ln(y, g_r[...], b_r[...]) + x
        if ne:
            y = y + refs[nx + 9][...]
        o_r[...] = y

    return pl.pallas_call(
        body, grid=(n // _BN,),
        in_specs=[_row_spec(lat)] * nx
        + [_row_spec(lat), _row_spec(lat), _full_spec(lat, lat),
           _full_spec(lat, lat), _full_spec(1, lat), _full_spec(lat, lat),
           _full_spec(1, lat), _full_spec(1, lat), _full_spec(1, lat)]
        + ([_row_spec(lat)] if ne else []),
        out_specs=_row_spec(lat),
        out_shape=jax.ShapeDtypeStruct((n, lat), F32))(*ins)


def _tc_combine(parts, mults, post=None):
    nc, n, d = parts.shape

    def body(*refs):
        in_r = refs[0]
        o_r = refs[-1]
        s = in_r[0] + in_r[1]
        if post == "inv":
            s = 1.0 / s
        elif post == "inveps":
            s = 1.0 / (s + 1e-12)
        for m_r in refs[1:-1]:
            s = s * m_r[...]
        o_r[...] = s

    return pl.pallas_call(
        body, grid=(n // _BN,),
        in_specs=[pl.BlockSpec((nc, _BN, d), lambda i: (0, i, 0))]
        + [_row_spec(d)] * len(mults),
        out_specs=_row_spec(d),
        out_shape=jax.ShapeDtypeStruct((n, d), F32))(parts, *mults)


def _tc_mul(a, b, col=None):
    n, d = a.shape
    ins = [a, b] + ([col] if col is not None else [])

    def body(*refs):
        o_r = refs[-1]
        y = refs[0][...] * refs[1][...]
        if col is not None:
            y = y * refs[2][...]
        o_r[...] = y

    return pl.pallas_call(
        body, grid=(n // _BN,),
        in_specs=[_row_spec(d), _row_spec(d)]
        + ([_row_spec(1)] if col is not None else []),
        out_specs=_row_spec(d),
        out_shape=jax.ShapeDtypeStruct((n, d), F32))(*ins)


def _tc_winmax(parts):
    p, n = parts.shape

    def body(in_r, w_r, m_r):
        m = jnp.max(in_r[...], axis=0, keepdims=True)
        w_r[...] = jnp.maximum(m, 0)
        m_r[...] = jnp.where(m >= 0, 1.0, 0.0).astype(F32)

    return pl.pallas_call(
        body, grid=(1,),
        in_specs=[pl.BlockSpec((p, n), lambda i: (0, 0))],
        out_specs=[pl.BlockSpec((1, n), lambda i: (0, 0))] * 2,
        out_shape=[jax.ShapeDtypeStruct((1, n), I32),
                   jax.ShapeDtypeStruct((1, n), F32)])(parts)


def _edge_w(p, lat):
    w0 = p["w0"]
    c3 = jnp.pad(w0[0:3], ((0, LAT - 3), (0, 0)))
    c4 = w0[3:4]
    wa = w0[4:4 + lat]
    wb = w0[4 + lat:4 + 2 * lat]
    return (c3, c4, wa, wb, p["b0"].reshape(1, -1), p["w1"],
            p["b1"].reshape(1, -1), p["ln_g"].reshape(1, -1),
            p["ln_b"].reshape(1, -1))


def _node_w(p, lat):
    return (p["w0"][:lat], p["w0"][lat:], p["b0"].reshape(1, -1), p["w1"],
            p["b1"].reshape(1, -1), p["ln_g"].reshape(1, -1),
            p["ln_b"].reshape(1, -1))


def _gmp(x_list, pi, pj, i_idx, j_idx, posk, pe, pn, n, lat, zz, extra=None):
    c3, c4, wa, wb, b0e, w1e, b1e, ge, be = _edge_w(pe, lat)
    a_t, b_t = _tc_prep(x_list, posk, wa, wb, c3)
    ga, gb = _sc_gather([a_t, b_t], [i_idx, j_idx])
    emb = _tc_edge(ga, gb, pi, pj, c4, b0e, w1e, b1e, ge, be)
    (agp,) = _sc_scatter([emb], j_idx, n, [zz])
    wx, wac, b0n, w1n, b1n, gn, bn = _node_w(pn, lat)
    return _tc_node(x_list, agp[0], agp[1], wx, wac, b0n, w1n, b1n, gn, bn,
                    extra=extra)


def kernel(h, m_ids, m_gs, pos, params):
    n, lat = h.shape
    e = m_gs.shape[-1]
    i0 = m_gs[0, 0]
    j0 = m_gs[0, 1]
    i1 = m_gs[1, 0]
    j1 = m_gs[1, 1]
    ids0 = m_ids[0]
    padn = ((n + NW * CH - 1) // (NW * CH)) * NW * CH

    posp = jnp.pad(pos, ((0, 0), (0, LAT - pos.shape[1])))
    zz128 = jnp.zeros((n, lat), F32)

    pi0, pj0 = _sc_gather([posp, posp], [i0, j0])
    h1 = _gmp([h], pi0, pj0, i0, j0, posp, params["down0"]["mlp_edge"],
              params["down0"]["mlp_node"], n, lat, zz128)

    (degp,) = _sc_scatter(None, i0, n, [zz128], ones_width=lat)
    nw128 = _tc_combine(degp, [], post="inv")
    awp = _sc_gather_scatter(nw128, i0, j0, n, zz128)
    iaw128 = _tc_combine(awp, [], post="inveps")

    hw1 = _tc_mul(h1, nw128)
    posw = _tc_mul(posp, nw128)
    shp = _sc_gather_scatter(hw1, i0, j0, n, zz128)
    spp = _sc_gather_scatter(posw, i0, j0, n, zz128)
    h2f = _tc_combine(shp, [iaw128])
    p2f = _tc_combine(spp, [iaw128])
    idp = jnp.pad(ids0, (0, padn - n))
    h2p, p2p = _sc_gather([h2f, p2f], [idp, idp])
    h2 = h2p[:n]
    p2 = p2p[:n]

    pi1, pj1 = _sc_gather([p2, p2], [i1, j1])
    h3 = _gmp([h2], pi1, pj1, i1, j1, p2, params["bottom"]["mlp_edge"],
              params["bottom"]["mlp_node"], n, lat, zz128)

    winp = _sc_win(idp, n, padn)
    winc, maskf = _tc_winmax(winp)
    wpad = jnp.pad(winc.reshape(-1)[:n], (0, padn - n))
    (h4p,) = _sc_gather([h3], [wpad])

    hw4 = _tc_mul(h4p[:n], iaw128, col=maskf.reshape(-1)[:n].reshape(n, 1))
    up_ = _sc_gather_scatter(hw4, j0, i0, n, zz128)
    hu = _tc_combine(up_, [nw128])

    out = _gmp([hu], pi0, pj0, i0, j0, posp, params["up0"]["mlp_edge"],
               params["up0"]["mlp_node"], n, lat, zz128, extra=h1)
    return out


# speedup vs baseline: 3.6745x; 1.0218x over previous
import functools

import jax
import jax.numpy as jnp
from jax import lax
from jax.experimental import pallas as pl
from jax.experimental.pallas import tpu as pltpu
from jax.experimental.pallas import tpu_sc as plsc

F32 = jnp.float32
I32 = jnp.int32
NC, NS = 2, 16
NW = NC * NS
CH = 40
LAT = 128


def _wid():
    return lax.axis_index("s") * NC + lax.axis_index("c")


def _mesh():
    return plsc.VectorSubcoreMesh(core_axis_name="c", subcore_axis_name="s")


def _pipe_depth(nch, p):
    for kd in (5, 4, 2):
        if nch % kd == 0 and kd * p <= 10:
            return kd
    return 1


def _sc_gather(tables, idxs):
    P = len(tables)
    B = idxs[0].shape[0]
    span = B // NW
    assert B % NW == 0 and span % CH == 0
    nch = span // CH
    K = _pipe_depth(nch, P)
    ng = nch // K
    out_type = tuple(jax.ShapeDtypeStruct((B, t.shape[1]), F32) for t in tables)
    scratch = []
    for t in tables:
        scratch += [pltpu.VMEM((K, CH), I32), pltpu.VMEM((K, CH, t.shape[1]), F32)]
    scratch += [pltpu.SemaphoreType.DMA, pltpu.SemaphoreType.DMA]

    @functools.partial(pl.kernel, out_type=out_type, mesh=_mesh(),
                       scratch_types=scratch)
    def k(*refs):
        t_refs = refs[:P]
        i_refs = refs[P:2 * P]
        o_refs = refs[2 * P:3 * P]
        sc = refs[3 * P:]
        gsem, wsem = sc[-2], sc[-1]
        base = _wid() * span

        def body(g, carry):
            for u in range(K):
                off = base + (g * K + u) * CH
                for p in range(P):
                    iv = sc[2 * p].at[u]
                    rv = sc[2 * p + 1].at[u]
                    dst = o_refs[p].at[pl.ds(off, CH)]

                    @pl.when(g > 0)
                    def _():
                        pltpu.make_async_copy(rv, dst, wsem).wait()

                    pltpu.sync_copy(i_refs[p].at[pl.ds(off, CH)], iv)
                    pltpu.async_copy(t_refs[p].at[iv], rv, gsem)
            for u in range(K):
                off = base + (g * K + u) * CH
                for p in range(P):
                    iv = sc[2 * p].at[u]
                    rv = sc[2 * p + 1].at[u]
                    pltpu.make_async_copy(t_refs[p].at[iv], rv, gsem).wait()
                    pltpu.async_copy(rv, o_refs[p].at[pl.ds(off, CH)], wsem)
            return carry

        lax.fori_loop(0, ng, body, 0)
        for u in range(K):
            off = base + (nch - K + u) * CH
            for p in range(P):
                rv = sc[2 * p + 1].at[u]
                pltpu.make_async_copy(rv, o_refs[p].at[pl.ds(off, CH)],
                                      wsem).wait()

    return k(*tables, *idxs)


def _sc_gather_diff(table, i_idx, j_idx):
    B = i_idx.shape[0]
    d = table.shape[1]
    span = B // NW
    assert B % NW == 0 and span % CH == 0
    nch = span // CH
    K = _pipe_depth(nch, 2)
    ng = nch // K

    @functools.partial(
        pl.kernel, out_type=jax.ShapeDtypeStruct((B, d), F32), mesh=_mesh(),
        scratch_types=[pltpu.VMEM((K, CH), I32), pltpu.VMEM((K, CH), I32),
                       pltpu.VMEM((K, CH, d), F32), pltpu.VMEM((K, CH, d), F32),
                       pltpu.SemaphoreType.DMA, pltpu.SemaphoreType.DMA])
    def k(t_ref, i_ref, j_ref, o_ref, ivb, jvb, b1, b2, gsem, wsem):
        base = _wid() * span

        def body(g, carry):
            for u in range(K):
                off = base + (g * K + u) * CH
                iv, jv = ivb.at[u], jvb.at[u]
                r1, r2 = b1.at[u], b2.at[u]

                @pl.when(g > 0)
                def _():
                    pltpu.make_async_copy(r1, o_ref.at[pl.ds(off, CH)],
                                          wsem).wait()

                pltpu.sync_copy(i_ref.at[pl.ds(off, CH)], iv)
                pltpu.sync_copy(j_ref.at[pl.ds(off, CH)], jv)
                pltpu.async_copy(t_ref.at[iv], r1, gsem)
                pltpu.async_copy(t_ref.at[jv], r2, gsem)
            for u in range(K):
                off = base + (g * K + u) * CH
                iv, jv = ivb.at[u], jvb.at[u]
                r1, r2 = b1.at[u], b2.at[u]
                pltpu.make_async_copy(t_ref.at[iv], r1, gsem).wait()
                pltpu.make_async_copy(t_ref.at[jv], r2, gsem).wait()
                for r in range(CH):
                    r1[r, pl.ds(0, 16)] = r1[r, pl.ds(0, 16)] - r2[r, pl.ds(0, 16)]
                pltpu.async_copy(r1, o_ref.at[pl.ds(off, CH)], wsem)
            return carry

        lax.fori_loop(0, ng, body, 0)
        for u in range(K):
            off = base + (nch - K + u) * CH
            pltpu.make_async_copy(b1.at[u], o_ref.at[pl.ds(off, CH)],
                                  wsem).wait()

    return k(table, i_idx, j_idx)


def _scatter_epilogue(sh, o_refs, cid, sid, rpt, rlast, P):
    @pl.when(sid < NS - 1)
    def _():
        for p in range(P):
            pltpu.sync_copy(sh[p].at[pl.ds(sid * rpt, rpt)],
                            o_refs[p].at[cid, pl.ds(sid * rpt, rpt)])

    @pl.when(sid == NS - 1)
    def _():
        for p in range(P):
            pltpu.sync_copy(sh[p].at[pl.ds((NS - 1) * rpt, rlast)],
                            o_refs[p].at[cid, pl.ds((NS - 1) * rpt, rlast)])


def _sc_scatter(vals, idx, n, zeros, ones_width=None):
    synth = ones_width is not None
    P = 1 if synth else len(vals)
    widths = [ones_width] if synth else [v.shape[1] for v in vals]
    e = idx.shape[0]
    spc = e // NC
    sps = spc // NS
    assert e % (NC * NS) == 0 and sps % CH == 0
    nch = sps // CH
    rpt = ((n // NS) // 8) * 8
    rlast = n - (NS - 1) * rpt
    out_type = tuple(jax.ShapeDtypeStruct((NC, n, d), F32) for d in widths)
    n_in = 0 if synth else P
    K = _pipe_depth(nch, P)
    ng = nch // K
    scratch = [pltpu.VMEM((K, CH), I32)]
    for d in widths:
        scratch.append(pltpu.VMEM((1 if synth else K, CH, d), F32))
    for d in widths:
        scratch.append(pltpu.VMEM_SHARED((n, d), F32))
    scratch += [pltpu.SemaphoreType.DMA, pltpu.SemaphoreType.DMA]

    @functools.partial(pl.kernel, out_type=out_type, mesh=_mesh(),
                       scratch_types=scratch)
    def k(*refs):
        v_refs = refs[:n_in]
        i_ref = refs[n_in]
        z_refs = refs[n_in + 1:n_in + 1 + P]
        o_refs = refs[n_in + 1 + P:n_in + 1 + 2 * P]
        ivb = refs[n_in + 1 + 2 * P]
        vb = refs[n_in + 2 + 2 * P:n_in + 2 + 3 * P]
        sh = refs[n_in + 2 + 3 * P:n_in + 2 + 4 * P]
        vsem, wsem = refs[-2], refs[-1]
        cid = lax.axis_index("c")
        sid = lax.axis_index("s")

        @pl.when(sid == 0)
        def _():
            for p in range(P):
                pltpu.sync_copy(z_refs[p], sh[p])

        if synth:
            def fill(r, carry):
                for s in range(widths[0] // 16):
                    vb[0][0, r, pl.ds(s * 16, 16)] = jnp.ones((16,), F32)
                return carry

            lax.fori_loop(0, CH, fill, 0)

        plsc.subcore_barrier()
        base = cid * spc + sid * sps

        def body(g, carry):
            for u in range(K):
                off = base + (g * K + u) * CH
                iv = ivb.at[u]
                for p in range(P):
                    bv = vb[p].at[0 if synth else u]

                    @pl.when(g > 0)
                    def _():
                        pltpu.make_async_copy(bv, sh[p].at[iv], wsem).wait()

                pltpu.sync_copy(i_ref.at[pl.ds(off, CH)], iv)
                if not synth:
                    for p in range(P):
                        pltpu.async_copy(v_refs[p].at[pl.ds(off, CH)],
                                         vb[p].at[u], vsem)
            for u in range(K):
                off = base + (g * K + u) * CH
                iv = ivb.at[u]
                for p in range(P):
                    bv = vb[p].at[0 if synth else u]
                    if not synth:
                        pltpu.make_async_copy(v_refs[p].at[pl.ds(off, CH)],
                                              bv, vsem).wait()
                    pltpu.async_copy(bv, sh[p].at[iv], wsem, add=True)
            return carry

        lax.fori_loop(0, ng, body, 0)
        for u in range(K):
            iv = ivb.at[u]
            for p in range(P):
                bv = vb[p].at[0 if synth else u]
                pltpu.make_async_copy(bv, sh[p].at[iv], wsem).wait()
        plsc.subcore_barrier()
        _scatter_epilogue(sh, o_refs, cid, sid, rpt, rlast, P)

    if synth:
        return k(idx, *zeros)
    return k(*vals, idx, *zeros)


def _sc_gather_scatter(table, gidx, sidx, n, zeros):
    d = table.shape[1]
    e = gidx.shape[0]
    spc = e // NC
    sps = spc // NS
    assert e % (NC * NS) == 0 and sps % CH == 0
    nch = sps // CH
    rpt = ((n // NS) // 8) * 8
    rlast = n - (NS - 1) * rpt

    K = _pipe_depth(nch, 1)
    ng = nch // K

    @functools.partial(
        pl.kernel, out_type=(jax.ShapeDtypeStruct((NC, n, d), F32),),
        mesh=_mesh(),
        scratch_types=[pltpu.VMEM((K, CH), I32), pltpu.VMEM((K, CH), I32),
                       pltpu.VMEM((K, CH, d), F32),
                       pltpu.VMEM_SHARED((n, d), F32),
                       pltpu.SemaphoreType.DMA, pltpu.SemaphoreType.DMA])
    def k(t_ref, g_ref, s_ref, z_ref, o_ref, gvb, svb, vb, sh, gsem, wsem):
        cid = lax.axis_index("c")
        sid = lax.axis_index("s")

        @pl.when(sid == 0)
        def _():
            pltpu.sync_copy(z_ref, sh)

        plsc.subcore_barrier()
        base = cid * spc + sid * sps

        def body(g, carry):
            for u in range(K):
                off = base + (g * K + u) * CH
                gv, sv, bv = gvb.at[u], svb.at[u], vb.at[u]

                @pl.when(g > 0)
                def _():
                    pltpu.make_async_copy(bv, sh.at[sv], wsem).wait()

                pltpu.sync_copy(g_ref.at[pl.ds(off, CH)], gv)
                pltpu.sync_copy(s_ref.at[pl.ds(off, CH)], sv)
                pltpu.async_copy(t_ref.at[gv], bv, gsem)
            for u in range(K):
                gv, sv, bv = gvb.at[u], svb.at[u], vb.at[u]
                pltpu.make_async_copy(t_ref.at[gv], bv, gsem).wait()
                pltpu.async_copy(bv, sh.at[sv], wsem, add=True)
            return carry

        lax.fori_loop(0, ng, body, 0)
        for u in range(K):
            pltpu.make_async_copy(vb.at[u], sh.at[svb.at[u]], wsem).wait()
        plsc.subcore_barrier()
        _scatter_epilogue([sh], [o_ref], cid, sid, rpt, rlast, 1)

    return k(table, gidx, sidx, zeros)[0]


def _sc_win(idp, n, padn):
    np2 = ((n + 127) // 128) * 128
    spw = padn // NW
    ng = spw // 16

    @functools.partial(
        pl.kernel, out_type=jax.ShapeDtypeStruct((NW * np2,), I32),
        mesh=_mesh(),
        scratch_types=[pltpu.VMEM((padn,), I32), pltpu.VMEM((np2 + 16,), I32)])
    def k(id_ref, o_ref, idv, acc):
        w = _wid()
        pltpu.sync_copy(id_ref, idv)

        def z(c, carry):
            acc[pl.ds(c * 16, 16)] = jnp.full((16,), -1, I32)
            return carry

        lax.fori_loop(0, (np2 + 16) // 16, z, 0)
        k0 = w * spw
        iot = lax.iota(I32, 16)

        def outer(g, carry):
            kbase = k0 + g * 16
            idvec = idv[pl.ds(kbase, 16)]
            for l in range(16):
                kk = kbase + l

                @pl.when(kk < n)
                def _():
                    t = idvec[l]
                    cur = acc[pl.ds(t, 16)]
                    acc[pl.ds(t, 16)] = jnp.where(iot == 0, kk, cur)
            return carry

        lax.fori_loop(0, ng, outer, 0)
        pltpu.sync_copy(acc.at[pl.ds(0, np2)], o_ref.at[pl.ds(w * np2, np2)])

    return k(idp).reshape(NW, np2)


_BN = 1000


def _row_spec(d):
    return pl.BlockSpec((_BN, d), lambda i: (i, 0))


def _full_spec(r, c):
    return pl.BlockSpec((r, c), lambda i: (0, 0))


def _tc_prep(x_list, posp, wa, wb, c3):
    n, lat = x_list[0].shape
    nx = len(x_list)

    def body(*refs):
        xs = refs[:nx]
        pp, wa_r, wb_r, c3_r, a_r, b_r = refs[nx:]
        x = xs[0][...]
        for r in xs[1:]:
            x = x + r[...]
        pc = jnp.dot(pp[...], c3_r[...], preferred_element_type=F32)
        a_r[...] = jnp.dot(x, wa_r[...], preferred_element_type=F32) + pc
        b_r[...] = jnp.dot(x, wb_r[...], preferred_element_type=F32) - pc

    return pl.pallas_call(
        body, grid=(n // _BN,),
        in_specs=[_row_spec(lat)] * nx + [_row_spec(LAT), _full_spec(lat, lat),
                                          _full_spec(lat, lat),
                                          _full_spec(LAT, lat)],
        out_specs=[_row_spec(lat)] * 2,
        out_shape=[jax.ShapeDtypeStruct((n, lat), F32)] * 2,
    )(*x_list, posp, wa, wb, c3)


def _ln(y, g, b):
    mu = jnp.mean(y, axis=-1, keepdims=True)
    yc = y - mu
    var = jnp.mean(yc * yc, axis=-1, keepdims=True)
    return yc * lax.rsqrt(var + 1e-5) * g + b


def _tc_edge(ga, gb, dd, c4, b0, w1, b1, g, b):
    e, lat = ga.shape

    def body(ga_r, gb_r, dd_r, c4_r, b0_r, w1_r, b1_r, g_r, b_r, o_r):
        d = dd_r[:, :16]
        nrm = jnp.sqrt(jnp.sum(d * d, axis=-1, keepdims=True) + 1e-12)
        y = ga_r[...] + gb_r[...] + nrm * c4_r[...] + b0_r[...]
        y = jnp.maximum(y, 0.0)
        y = jnp.dot(y, w1_r[...], preferred_element_type=F32) + b1_r[...]
        o_r[...] = _ln(y, g_r[...], b_r[...])

    return pl.pallas_call(
        body, grid=(e // _BN,),
        in_specs=[_row_spec(lat)] * 3
        + [_full_spec(1, lat), _full_spec(1, lat), _full_spec(lat, lat),
           _full_spec(1, lat), _full_spec(1, lat), _full_spec(1, lat)],
        out_specs=_row_spec(lat),
        out_shape=jax.ShapeDtypeStruct((e, lat), F32))(
            ga, gb, dd, c4, b0, w1, b1, g, b)


def _tc_node(x_list, a0, a1, wx, wa, b0, w1, b1, g, b, extra=None):
    n, lat = x_list[0].shape
    nx = len(x_list)
    ins = list(x_list) + [a0, a1, wx, wa, b0, w1, b1, g, b]
    if extra is not None:
        ins.append(extra)
    ne = extra is not None

    def body(*refs):
        xs = refs[:nx]
        a0_r, a1_r, wx_r, wa_r, b0_r, w1_r, b1_r, g_r, b_r = refs[nx:nx + 9]
        o_r = refs[-1]
        x = xs[0][...]
        for r in xs[1:]:
            x = x + r[...]
        ag = a0_r[...] + a1_r[...]
        y = (jnp.dot(x, wx_r[...], preferred_element_type=F32)
             + jnp.dot(ag, wa_r[...], preferred_element_type=F32) + b0_r[...])
        y = jnp.maximum(y, 0.0)
        y = jnp.dot(y, w1_r[...], preferred_element_type=F32) + b1_r[...]
        y = _ln(y, g_r[...], b_r[...]) + x
        if ne:
            y = y + refs[nx + 9][...]
        o_r[...] = y

    return pl.pallas_call(
        body, grid=(n // _BN,),
        in_specs=[_row_spec(lat)] * nx
        + [_row_spec(lat), _row_spec(lat), _full_spec(lat, lat),
           _full_spec(lat, lat), _full_spec(1, lat), _full_spec(lat, lat),
           _full_spec(1, lat), _full_spec(1, lat), _full_spec(1, lat)]
        + ([_row_spec(lat)] if ne else []),
        out_specs=_row_spec(lat),
        out_shape=jax.ShapeDtypeStruct((n, lat), F32))(*ins)


def _tc_combine(parts, mults, post=None):
    nc, n, d = parts.shape

    def body(*refs):
        in_r = refs[0]
        o_r = refs[-1]
        s = in_r[0] + in_r[1]
        if post == "inv":
            s = 1.0 / s
        elif post == "inveps":
            s = 1.0 / (s + 1e-12)
        for m_r in refs[1:-1]:
            s = s * m_r[...]
        o_r[...] = s

    return pl.pallas_call(
        body, grid=(n // _BN,),
        in_specs=[pl.BlockSpec((nc, _BN, d), lambda i: (0, i, 0))]
        + [_row_spec(d)] * len(mults),
        out_specs=_row_spec(d),
        out_shape=jax.ShapeDtypeStruct((n, d), F32))(parts, *mults)


def _tc_mul(a, b, col=None):
    n, d = a.shape
    ins = [a, b] + ([col] if col is not None else [])

    def body(*refs):
        o_r = refs[-1]
        y = refs[0][...] * refs[1][...]
        if col is not None:
            y = y * refs[2][...]
        o_r[...] = y

    return pl.pallas_call(
        body, grid=(n // _BN,),
        in_specs=[_row_spec(d), _row_spec(d)]
        + ([_row_spec(1)] if col is not None else []),
        out_specs=_row_spec(d),
        out_shape=jax.ShapeDtypeStruct((n, d), F32))(*ins)


def _tc_winmax(parts):
    p, n = parts.shape

    def body(in_r, w_r, m_r):
        m = jnp.max(in_r[...], axis=0, keepdims=True)
        w_r[...] = jnp.maximum(m, 0)
        m_r[...] = jnp.where(m >= 0, 1.0, 0.0).astype(F32)

    return pl.pallas_call(
        body, grid=(1,),
        in_specs=[pl.BlockSpec((p, n), lambda i: (0, 0))],
        out_specs=[pl.BlockSpec((1, n), lambda i: (0, 0))] * 2,
        out_shape=[jax.ShapeDtypeStruct((1, n), I32),
                   jax.ShapeDtypeStruct((1, n), F32)])(parts)


def _edge_w(p, lat):
    w0 = p["w0"]
    c3 = jnp.pad(w0[0:3], ((0, LAT - 3), (0, 0)))
    c4 = w0[3:4]
    wa = w0[4:4 + lat]
    wb = w0[4 + lat:4 + 2 * lat]
    return (c3, c4, wa, wb, p["b0"].reshape(1, -1), p["w1"],
            p["b1"].reshape(1, -1), p["ln_g"].reshape(1, -1),
            p["ln_b"].reshape(1, -1))


def _node_w(p, lat):
    return (p["w0"][:lat], p["w0"][lat:], p["b0"].reshape(1, -1), p["w1"],
            p["b1"].reshape(1, -1), p["ln_g"].reshape(1, -1),
            p["ln_b"].reshape(1, -1))


def _gmp(x_list, dd, i_idx, j_idx, posk, pe, pn, n, lat, zz, extra=None):
    c3, c4, wa, wb, b0e, w1e, b1e, ge, be = _edge_w(pe, lat)
    a_t, b_t = _tc_prep(x_list, posk, wa, wb, c3)
    ga, gb = _sc_gather([a_t, b_t], [i_idx, j_idx])
    emb = _tc_edge(ga, gb, dd, c4, b0e, w1e, b1e, ge, be)
    (agp,) = _sc_scatter([emb], j_idx, n, [zz])
    wx, wac, b0n, w1n, b1n, gn, bn = _node_w(pn, lat)
    return _tc_node(x_list, agp[0], agp[1], wx, wac, b0n, w1n, b1n, gn, bn,
                    extra=extra)


def kernel(h, m_ids, m_gs, pos, params):
    n, lat = h.shape
    e = m_gs.shape[-1]
    i0 = m_gs[0, 0]
    j0 = m_gs[0, 1]
    i1 = m_gs[1, 0]
    j1 = m_gs[1, 1]
    ids0 = m_ids[0]
    padn = ((n + NW * CH - 1) // (NW * CH)) * NW * CH

    posp = jnp.pad(pos, ((0, 0), (0, LAT - pos.shape[1])))
    zz128 = jnp.zeros((n, lat), F32)

    d0 = _sc_gather_diff(posp, i0, j0)
    h1 = _gmp([h], d0, i0, j0, posp, params["down0"]["mlp_edge"],
              params["down0"]["mlp_node"], n, lat, zz128)

    (degp,) = _sc_scatter(None, i0, n, [zz128], ones_width=lat)
    nw128 = _tc_combine(degp, [], post="inv")
    awp = _sc_gather_scatter(nw128, i0, j0, n, zz128)
    iaw128 = _tc_combine(awp, [], post="inveps")

    hw1 = _tc_mul(h1, nw128)
    posw = _tc_mul(posp, nw128)
    shp = _sc_gather_scatter(hw1, i0, j0, n, zz128)
    spp = _sc_gather_scatter(posw, i0, j0, n, zz128)
    h2f = _tc_combine(shp, [iaw128])
    p2f = _tc_combine(spp, [iaw128])
    idp = jnp.pad(ids0, (0, padn - n))
    h2p, p2p = _sc_gather([h2f, p2f], [idp, idp])
    h2 = h2p[:n]
    p2 = p2p[:n]

    d1 = _sc_gather_diff(p2, i1, j1)
    h3 = _gmp([h2], d1, i1, j1, p2, params["bottom"]["mlp_edge"],
              params["bottom"]["mlp_node"], n, lat, zz128)

    winp = _sc_win(idp, n, padn)
    winc, maskf = _tc_winmax(winp)
    wpad = jnp.pad(winc.reshape(-1)[:n], (0, padn - n))
    (h4p,) = _sc_gather([h3], [wpad])

    hw4 = _tc_mul(h4p[:n], iaw128, col=maskf.reshape(-1)[:n].reshape(n, 1))
    up_ = _sc_gather_scatter(hw4, j0, i0, n, zz128)
    hu = _tc_combine(up_, [nw128])

    out = _gmp([hu], d0, i0, j0, posp, params["up0"]["mlp_edge"],
               params["up0"]["mlp_node"], n, lat, zz128, extra=h1)
    return out


# speedup vs baseline: 3.7291x; 1.0148x over previous
import functools

import jax
import jax.numpy as jnp
from jax import lax
from jax.experimental import pallas as pl
from jax.experimental.pallas import tpu as pltpu
from jax.experimental.pallas import tpu_sc as plsc

F32 = jnp.float32
I32 = jnp.int32
NC, NS = 2, 16
NW = NC * NS
CH = 40
LAT = 128


def _wid():
    return lax.axis_index("s") * NC + lax.axis_index("c")


def _mesh():
    return plsc.VectorSubcoreMesh(core_axis_name="c", subcore_axis_name="s")


def _pipe_depth(nch, p):
    for kd in (5, 4, 2):
        if nch % kd == 0 and kd * p <= 10:
            return kd
    return 1


def _sc_gather(tables, idxs):
    P = len(tables)
    B = idxs[0].shape[0]
    span = B // NW
    assert B % NW == 0 and span % CH == 0
    nch = span // CH
    K = _pipe_depth(nch, P)
    ng = nch // K
    out_type = tuple(jax.ShapeDtypeStruct((B, t.shape[1]), t.dtype)
                     for t in tables)
    scratch = []
    for t in tables:
        scratch += [pltpu.VMEM((K, CH), I32),
                    pltpu.VMEM((K, CH, t.shape[1]), t.dtype)]
    scratch += [pltpu.SemaphoreType.DMA, pltpu.SemaphoreType.DMA]

    @functools.partial(pl.kernel, out_type=out_type, mesh=_mesh(),
                       scratch_types=scratch)
    def k(*refs):
        t_refs = refs[:P]
        i_refs = refs[P:2 * P]
        o_refs = refs[2 * P:3 * P]
        sc = refs[3 * P:]
        gsem, wsem = sc[-2], sc[-1]
        base = _wid() * span

        def body(g, carry):
            for u in range(K):
                off = base + (g * K + u) * CH
                for p in range(P):
                    iv = sc[2 * p].at[u]
                    rv = sc[2 * p + 1].at[u]
                    dst = o_refs[p].at[pl.ds(off, CH)]

                    @pl.when(g > 0)
                    def _():
                        pltpu.make_async_copy(rv, dst, wsem).wait()

                    pltpu.sync_copy(i_refs[p].at[pl.ds(off, CH)], iv)
                    pltpu.async_copy(t_refs[p].at[iv], rv, gsem)
            for u in range(K):
                off = base + (g * K + u) * CH
                for p in range(P):
                    iv = sc[2 * p].at[u]
                    rv = sc[2 * p + 1].at[u]
                    pltpu.make_async_copy(t_refs[p].at[iv], rv, gsem).wait()
                    pltpu.async_copy(rv, o_refs[p].at[pl.ds(off, CH)], wsem)
            return carry

        lax.fori_loop(0, ng, body, 0)
        for u in range(K):
            off = base + (nch - K + u) * CH
            for p in range(P):
                rv = sc[2 * p + 1].at[u]
                pltpu.make_async_copy(rv, o_refs[p].at[pl.ds(off, CH)],
                                      wsem).wait()

    return k(*tables, *idxs)


def _sc_gather_combine(table_i, table_j, i_idx, j_idx, mode):
    B = i_idx.shape[0]
    d = table_i.shape[1]
    span = B // NW
    assert B % NW == 0 and span % CH == 0
    nch = span // CH
    K = _pipe_depth(nch, 2)
    ng = nch // K

    @functools.partial(
        pl.kernel, out_type=jax.ShapeDtypeStruct((B, d), F32), mesh=_mesh(),
        scratch_types=[pltpu.VMEM((K, CH), I32), pltpu.VMEM((K, CH), I32),
                       pltpu.VMEM((K, CH, d), F32), pltpu.VMEM((K, CH, d), F32),
                       pltpu.SemaphoreType.DMA, pltpu.SemaphoreType.DMA])
    def k(ti_ref, tj_ref, i_ref, j_ref, o_ref, ivb, jvb, b1, b2, gsem, wsem):
        base = _wid() * span

        def body(g, carry):
            for u in range(K):
                off = base + (g * K + u) * CH
                iv, jv = ivb.at[u], jvb.at[u]
                r1, r2 = b1.at[u], b2.at[u]

                @pl.when(g > 0)
                def _():
                    pltpu.make_async_copy(r1, o_ref.at[pl.ds(off, CH)],
                                          wsem).wait()

                pltpu.sync_copy(i_ref.at[pl.ds(off, CH)], iv)
                pltpu.sync_copy(j_ref.at[pl.ds(off, CH)], jv)
                pltpu.async_copy(ti_ref.at[iv], r1, gsem)
                pltpu.async_copy(tj_ref.at[jv], r2, gsem)
            for u in range(K):
                off = base + (g * K + u) * CH
                iv, jv = ivb.at[u], jvb.at[u]
                r1, r2 = b1.at[u], b2.at[u]
                pltpu.make_async_copy(ti_ref.at[iv], r1, gsem).wait()
                pltpu.make_async_copy(tj_ref.at[jv], r2, gsem).wait()
                if mode == "diff16":
                    for r in range(CH):
                        r1[r, pl.ds(0, 16)] = (r1[r, pl.ds(0, 16)]
                                               - r2[r, pl.ds(0, 16)])
                else:
                    def rowadd(r, c2):
                        for s in range(d // 16):
                            sl = pl.ds(s * 16, 16)
                            r1[r, sl] = r1[r, sl] + r2[r, sl]
                        return c2

                    lax.fori_loop(0, CH, rowadd, 0)
                pltpu.async_copy(r1, o_ref.at[pl.ds(off, CH)], wsem)
            return carry

        lax.fori_loop(0, ng, body, 0)
        for u in range(K):
            off = base + (nch - K + u) * CH
            pltpu.make_async_copy(b1.at[u], o_ref.at[pl.ds(off, CH)],
                                  wsem).wait()

    return k(table_i, table_j, i_idx, j_idx)


def _scatter_epilogue(sh, o_refs, cid, sid, rpt, rlast, P):
    @pl.when(sid < NS - 1)
    def _():
        for p in range(P):
            pltpu.sync_copy(sh[p].at[pl.ds(sid * rpt, rpt)],
                            o_refs[p].at[cid, pl.ds(sid * rpt, rpt)])

    @pl.when(sid == NS - 1)
    def _():
        for p in range(P):
            pltpu.sync_copy(sh[p].at[pl.ds((NS - 1) * rpt, rlast)],
                            o_refs[p].at[cid, pl.ds((NS - 1) * rpt, rlast)])


def _sc_scatter(vals, idx, n, zeros, ones_width=None):
    synth = ones_width is not None
    P = 1 if synth else len(vals)
    widths = [ones_width] if synth else [v.shape[1] for v in vals]
    e = idx.shape[0]
    spc = e // NC
    sps = spc // NS
    assert e % (NC * NS) == 0 and sps % CH == 0
    nch = sps // CH
    rpt = ((n // NS) // 8) * 8
    rlast = n - (NS - 1) * rpt
    out_type = tuple(jax.ShapeDtypeStruct((NC, n, d), F32) for d in widths)
    n_in = 0 if synth else P
    K = _pipe_depth(nch, P)
    ng = nch // K
    scratch = [pltpu.VMEM((K, CH), I32)]
    for d in widths:
        scratch.append(pltpu.VMEM((1 if synth else K, CH, d), F32))
    for d in widths:
        scratch.append(pltpu.VMEM_SHARED((n, d), F32))
    scratch += [pltpu.SemaphoreType.DMA, pltpu.SemaphoreType.DMA]

    @functools.partial(pl.kernel, out_type=out_type, mesh=_mesh(),
                       scratch_types=scratch)
    def k(*refs):
        v_refs = refs[:n_in]
        i_ref = refs[n_in]
        z_refs = refs[n_in + 1:n_in + 1 + P]
        o_refs = refs[n_in + 1 + P:n_in + 1 + 2 * P]
        ivb = refs[n_in + 1 + 2 * P]
        vb = refs[n_in + 2 + 2 * P:n_in + 2 + 3 * P]
        sh = refs[n_in + 2 + 3 * P:n_in + 2 + 4 * P]
        vsem, wsem = refs[-2], refs[-1]
        cid = lax.axis_index("c")
        sid = lax.axis_index("s")

        @pl.when(sid == 0)
        def _():
            for p in range(P):
                pltpu.sync_copy(z_refs[p], sh[p])

        if synth:
            def fill(r, carry):
                for s in range(widths[0] // 16):
                    vb[0][0, r, pl.ds(s * 16, 16)] = jnp.ones((16,), F32)
                return carry

            lax.fori_loop(0, CH, fill, 0)

        plsc.subcore_barrier()
        base = cid * spc + sid * sps

        def body(g, carry):
            for u in range(K):
                off = base + (g * K + u) * CH
                iv = ivb.at[u]
                for p in range(P):
                    bv = vb[p].at[0 if synth else u]

                    @pl.when(g > 0)
                    def _():
                        pltpu.make_async_copy(bv, sh[p].at[iv], wsem).wait()

                pltpu.sync_copy(i_ref.at[pl.ds(off, CH)], iv)
                if not synth:
                    for p in range(P):
                        pltpu.async_copy(v_refs[p].at[pl.ds(off, CH)],
                                         vb[p].at[u], vsem)
            for u in range(K):
                off = base + (g * K + u) * CH
                iv = ivb.at[u]
                for p in range(P):
                    bv = vb[p].at[0 if synth else u]
                    if not synth:
                        pltpu.make_async_copy(v_refs[p].at[pl.ds(off, CH)],
                                              bv, vsem).wait()
                    pltpu.async_copy(bv, sh[p].at[iv], wsem, add=True)
            return carry

        lax.fori_loop(0, ng, body, 0)
        for u in range(K):
            iv = ivb.at[u]
            for p in range(P):
                bv = vb[p].at[0 if synth else u]
                pltpu.make_async_copy(bv, sh[p].at[iv], wsem).wait()
        plsc.subcore_barrier()
        _scatter_epilogue(sh, o_refs, cid, sid, rpt, rlast, P)

    if synth:
        return k(idx, *zeros)
    return k(*vals, idx, *zeros)


def _sc_gather_scatter(table, gidx, sidx, n, zeros):
    d = table.shape[1]
    e = gidx.shape[0]
    spc = e // NC
    sps = spc // NS
    assert e % (NC * NS) == 0 and sps % CH == 0
    nch = sps // CH
    rpt = ((n // NS) // 8) * 8
    rlast = n - (NS - 1) * rpt

    K = _pipe_depth(nch, 1)
    ng = nch // K

    @functools.partial(
        pl.kernel, out_type=(jax.ShapeDtypeStruct((NC, n, d), F32),),
        mesh=_mesh(),
        scratch_types=[pltpu.VMEM((K, CH), I32), pltpu.VMEM((K, CH), I32),
                       pltpu.VMEM((K, CH, d), F32),
                       pltpu.VMEM_SHARED((n, d), F32),
                       pltpu.SemaphoreType.DMA, pltpu.SemaphoreType.DMA])
    def k(t_ref, g_ref, s_ref, z_ref, o_ref, gvb, svb, vb, sh, gsem, wsem):
        cid = lax.axis_index("c")
        sid = lax.axis_index("s")

        @pl.when(sid == 0)
        def _():
            pltpu.sync_copy(z_ref, sh)

        plsc.subcore_barrier()
        base = cid * spc + sid * sps

        def body(g, carry):
            for u in range(K):
                off = base + (g * K + u) * CH
                gv, sv, bv = gvb.at[u], svb.at[u], vb.at[u]

                @pl.when(g > 0)
                def _():
                    pltpu.make_async_copy(bv, sh.at[sv], wsem).wait()

                pltpu.sync_copy(g_ref.at[pl.ds(off, CH)], gv)
                pltpu.sync_copy(s_ref.at[pl.ds(off, CH)], sv)
                pltpu.async_copy(t_ref.at[gv], bv, gsem)
            for u in range(K):
                gv, sv, bv = gvb.at[u], svb.at[u], vb.at[u]
                pltpu.make_async_copy(t_ref.at[gv], bv, gsem).wait()
                pltpu.async_copy(bv, sh.at[sv], wsem, add=True)
            return carry

        lax.fori_loop(0, ng, body, 0)
        for u in range(K):
            pltpu.make_async_copy(vb.at[u], sh.at[svb.at[u]], wsem).wait()
        plsc.subcore_barrier()
        _scatter_epilogue([sh], [o_ref], cid, sid, rpt, rlast, 1)

    return k(table, gidx, sidx, zeros)[0]


def _sc_win(idp, n, padn):
    np2 = ((n + 127) // 128) * 128
    spw = padn // NW
    ng = spw // 16

    @functools.partial(
        pl.kernel, out_type=jax.ShapeDtypeStruct((NW * np2,), I32),
        mesh=_mesh(),
        scratch_types=[pltpu.VMEM((padn,), I32), pltpu.VMEM((np2 + 16,), I32)])
    def k(id_ref, o_ref, idv, acc):
        w = _wid()
        pltpu.sync_copy(id_ref, idv)

        def z(c, carry):
            acc[pl.ds(c * 16, 16)] = jnp.full((16,), -1, I32)
            return carry

        lax.fori_loop(0, (np2 + 16) // 16, z, 0)
        k0 = w * spw
        iot = lax.iota(I32, 16)

        def outer(g, carry):
            kbase = k0 + g * 16
            idvec = idv[pl.ds(kbase, 16)]
            for l in range(16):
                kk = kbase + l

                @pl.when(kk < n)
                def _():
                    t = idvec[l]
                    cur = acc[pl.ds(t, 16)]
                    acc[pl.ds(t, 16)] = jnp.where(iot == 0, kk, cur)
            return carry

        lax.fori_loop(0, ng, outer, 0)
        pltpu.sync_copy(acc.at[pl.ds(0, np2)], o_ref.at[pl.ds(w * np2, np2)])

    return k(idp).reshape(NW, np2)


_BN = 1000


def _row_spec(d):
    return pl.BlockSpec((_BN, d), lambda i: (i, 0))


def _full_spec(r, c):
    return pl.BlockSpec((r, c), lambda i: (0, 0))


def _tc_prep(x_list, posp, wa, wb, c3):
    n, lat = x_list[0].shape
    nx = len(x_list)

    def body(*refs):
        xs = refs[:nx]
        pp, wa_r, wb_r, c3_r, a_r, b_r = refs[nx:]
        x = xs[0][...]
        for r in xs[1:]:
            x = x + r[...]
        pc = jnp.dot(pp[...], c3_r[...], preferred_element_type=F32)
        a_r[...] = jnp.dot(x, wa_r[...], preferred_element_type=F32) + pc
        b_r[...] = jnp.dot(x, wb_r[...], preferred_element_type=F32) - pc

    return pl.pallas_call(
        body, grid=(n // _BN,),
        in_specs=[_row_spec(lat)] * nx + [_row_spec(LAT), _full_spec(lat, lat),
                                          _full_spec(lat, lat),
                                          _full_spec(LAT, lat)],
        out_specs=[_row_spec(lat)] * 2,
        out_shape=[jax.ShapeDtypeStruct((n, lat), F32)] * 2,
    )(*x_list, posp, wa, wb, c3)


def _ln(y, g, b):
    mu = jnp.mean(y, axis=-1, keepdims=True)
    yc = y - mu
    var = jnp.mean(yc * yc, axis=-1, keepdims=True)
    return yc * lax.rsqrt(var + 1e-5) * g + b


def _tc_edge(gab, dd, c4, b0, w1, b1, g, b):
    e, lat = gab.shape

    def body(gab_r, dd_r, c4_r, b0_r, w1_r, b1_r, g_r, b_r, o_r):
        d = dd_r[:, :16]
        nrm = jnp.sqrt(jnp.sum(d * d, axis=-1, keepdims=True) + 1e-12)
        y = gab_r[...] + nrm * c4_r[...] + b0_r[...]
        y = jnp.maximum(y, 0.0)
        y = jnp.dot(y, w1_r[...], preferred_element_type=F32) + b1_r[...]
        o_r[...] = _ln(y, g_r[...], b_r[...])

    return pl.pallas_call(
        body, grid=(e // _BN,),
        in_specs=[_row_spec(lat)] * 2
        + [_full_spec(1, lat), _full_spec(1, lat), _full_spec(lat, lat),
           _full_spec(1, lat), _full_spec(1, lat), _full_spec(1, lat)],
        out_specs=_row_spec(lat),
        out_shape=jax.ShapeDtypeStruct((e, lat), F32))(
            gab, dd, c4, b0, w1, b1, g, b)


def _tc_node(x_list, a0, a1, wx, wa, b0, w1, b1, g, b, extra=None):
    n, lat = x_list[0].shape
    nx = len(x_list)
    ins = list(x_list) + [a0, a1, wx, wa, b0, w1, b1, g, b]
    if extra is not None:
        ins.append(extra)
    ne = extra is not None

    def body(*refs):
        xs = refs[:nx]
        a0_r, a1_r, wx_r, wa_r, b0_r, w1_r, b1_r, g_r, b_r = refs[nx:nx + 9]
        o_r = refs[-1]
        x = xs[0][...]
        for r in xs[1:]:
            x = x + r[...]
        ag = a0_r[...] + a1_r[...]
        y = (jnp.dot(x, wx_r[...], preferred_element_type=F32)
             + jnp.dot(ag, wa_r[...], preferred_element_type=F32) + b0_r[...])
        y = jnp.maximum(y, 0.0)
        y = jnp.dot(y, w1_r[...], preferred_element_type=F32) + b1_r[...]
        y = _ln(y, g_r[...], b_r[...]) + x
        if ne:
            y = y + refs[nx + 9][...]
        o_r[...] = y

    return pl.pallas_call(
        body, grid=(n // _BN,),
        in_specs=[_row_spec(lat)] * nx
        + [_row_spec(lat), _row_spec(lat), _full_spec(lat, lat),
           _full_spec(lat, lat), _full_spec(1, lat), _full_spec(lat, lat),
           _full_spec(1, lat), _full_spec(1, lat), _full_spec(1, lat)]
        + ([_row_spec(lat)] if ne else []),
        out_specs=_row_spec(lat),
        out_shape=jax.ShapeDtypeStruct((n, lat), F32))(*ins)


def _tc_combine(parts, mults, post=None):
    nc, n, d = parts.shape

    def body(*refs):
        in_r = refs[0]
        o_r = refs[-1]
        s = in_r[0] + in_r[1]
        if post == "inv":
            s = 1.0 / s
        elif post == "inveps":
            s = 1.0 / (s + 1e-12)
        for m_r in refs[1:-1]:
            s = s * m_r[...]
        o_r[...] = s

    return pl.pallas_call(
        body, grid=(n // _BN,),
        in_specs=[pl.BlockSpec((nc, _BN, d), lambda i: (0, i, 0))]
        + [_row_spec(d)] * len(mults),
        out_specs=_row_spec(d),
        out_shape=jax.ShapeDtypeStruct((n, d), F32))(parts, *mults)


def _tc_mul(a, b, col=None):
    n, d = a.shape
    ins = [a, b] + ([col] if col is not None else [])

    def body(*refs):
        o_r = refs[-1]
        y = refs[0][...] * refs[1][...]
        if col is not None:
            y = y * refs[2][...]
        o_r[...] = y

    return pl.pallas_call(
        body, grid=(n // _BN,),
        in_specs=[_row_spec(d), _row_spec(d)]
        + ([_row_spec(1)] if col is not None else []),
        out_specs=_row_spec(d),
        out_shape=jax.ShapeDtypeStruct((n, d), F32))(*ins)


def _tc_winmax(parts):
    p, n = parts.shape

    def body(in_r, w_r, m_r):
        m = jnp.max(in_r[...], axis=0, keepdims=True)
        w_r[...] = jnp.maximum(m, 0)
        m_r[...] = jnp.where(m >= 0, 1.0, 0.0).astype(F32)

    return pl.pallas_call(
        body, grid=(1,),
        in_specs=[pl.BlockSpec((p, n), lambda i: (0, 0))],
        out_specs=[pl.BlockSpec((1, n), lambda i: (0, 0))] * 2,
        out_shape=[jax.ShapeDtypeStruct((1, n), I32),
                   jax.ShapeDtypeStruct((1, n), F32)])(parts)


def _edge_w(p, lat):
    w0 = p["w0"]
    c3 = jnp.pad(w0[0:3], ((0, LAT - 3), (0, 0)))
    c4 = w0[3:4]
    wa = w0[4:4 + lat]
    wb = w0[4 + lat:4 + 2 * lat]
    return (c3, c4, wa, wb, p["b0"].reshape(1, -1), p["w1"],
            p["b1"].reshape(1, -1), p["ln_g"].reshape(1, -1),
            p["ln_b"].reshape(1, -1))


def _node_w(p, lat):
    return (p["w0"][:lat], p["w0"][lat:], p["b0"].reshape(1, -1), p["w1"],
            p["b1"].reshape(1, -1), p["ln_g"].reshape(1, -1),
            p["ln_b"].reshape(1, -1))


def _gmp(x_list, dd, i_idx, j_idx, posk, pe, pn, n, lat, zz, extra=None):
    c3, c4, wa, wb, b0e, w1e, b1e, ge, be = _edge_w(pe, lat)
    a_t, b_t = _tc_prep(x_list, posk, wa, wb, c3)
    gab = _sc_gather_combine(a_t, b_t, i_idx, j_idx, "add")
    emb = _tc_edge(gab, dd, c4, b0e, w1e, b1e, ge, be)
    (agp,) = _sc_scatter([emb], j_idx, n, [zz])
    wx, wac, b0n, w1n, b1n, gn, bn = _node_w(pn, lat)
    return _tc_node(x_list, agp[0], agp[1], wx, wac, b0n, w1n, b1n, gn, bn,
                    extra=extra)


def kernel(h, m_ids, m_gs, pos, params):
    n, lat = h.shape
    e = m_gs.shape[-1]
    i0 = m_gs[0, 0]
    j0 = m_gs[0, 1]
    i1 = m_gs[1, 0]
    j1 = m_gs[1, 1]
    ids0 = m_ids[0]
    padn = ((n + NW * CH - 1) // (NW * CH)) * NW * CH

    posp = jnp.pad(pos, ((0, 0), (0, LAT - pos.shape[1])))
    zz128 = jnp.zeros((n, lat), F32)

    d0 = _sc_gather_combine(posp, posp, i0, j0, "diff16")
    h1 = _gmp([h], d0, i0, j0, posp, params["down0"]["mlp_edge"],
              params["down0"]["mlp_node"], n, lat, zz128)

    (degp,) = _sc_scatter(None, i0, n, [zz128], ones_width=lat)
    nw128 = _tc_combine(degp, [], post="inv")
    awp = _sc_gather_scatter(nw128, i0, j0, n, zz128)
    iaw128 = _tc_combine(awp, [], post="inveps")

    hw1 = _tc_mul(h1, nw128)
    posw = _tc_mul(posp, nw128)
    shp = _sc_gather_scatter(hw1, i0, j0, n, zz128)
    spp = _sc_gather_scatter(posw, i0, j0, n, zz128)
    h2f = _tc_combine(shp, [iaw128])
    p2f = _tc_combine(spp, [iaw128])
    idp = jnp.pad(ids0, (0, padn - n))
    h2p, p2p = _sc_gather([h2f, p2f], [idp, idp])
    h2 = h2p[:n]
    p2 = p2p[:n]

    d1 = _sc_gather_combine(p2, p2, i1, j1, "diff16")
    h3 = _gmp([h2], d1, i1, j1, p2, params["bottom"]["mlp_edge"],
              params["bottom"]["mlp_node"], n, lat, zz128)

    winp = _sc_win(idp, n, padn)
    winc, maskf = _tc_winmax(winp)
    wpad = jnp.pad(winc.reshape(-1)[:n], (0, padn - n))
    (h4p,) = _sc_gather([h3], [wpad])

    hw4 = _tc_mul(h4p[:n], iaw128, col=maskf.reshape(-1)[:n].reshape(n, 1))
    up_ = _sc_gather_scatter(hw4, j0, i0, n, zz128)
    hu = _tc_combine(up_, [nw128])

    out = _gmp([hu], d0, i0, j0, posp, params["up0"]["mlp_edge"],
               params["up0"]["mlp_node"], n, lat, zz128, extra=h1)
    return out


# speedup vs baseline: 4.0136x; 1.0763x over previous
import functools

import jax
import jax.numpy as jnp
from jax import lax
from jax.experimental import pallas as pl
from jax.experimental.pallas import tpu as pltpu
from jax.experimental.pallas import tpu_sc as plsc

F32 = jnp.float32
I32 = jnp.int32
NC, NS = 2, 16
NW = NC * NS
CH = 40
LAT = 128


def _wid():
    return lax.axis_index("s") * NC + lax.axis_index("c")


def _mesh():
    return plsc.VectorSubcoreMesh(core_axis_name="c", subcore_axis_name="s")


def _pipe_depth(nch, p):
    for kd in (5, 4, 2):
        if nch % kd == 0 and kd * p <= 10:
            return kd
    return 1


def _sc_gather(tables, idxs):
    P = len(tables)
    B = idxs[0].shape[0]
    span = B // NW
    assert B % NW == 0 and span % CH == 0
    nch = span // CH
    K = _pipe_depth(nch, P)
    ng = nch // K
    out_type = tuple(jax.ShapeDtypeStruct((B, t.shape[1]), t.dtype)
                     for t in tables)
    scratch = []
    for t in tables:
        scratch += [pltpu.VMEM((K, CH), I32),
                    pltpu.VMEM((K, CH, t.shape[1]), t.dtype)]
    scratch += [pltpu.SemaphoreType.DMA, pltpu.SemaphoreType.DMA]

    @functools.partial(pl.kernel, out_type=out_type, mesh=_mesh(),
                       scratch_types=scratch)
    def k(*refs):
        t_refs = refs[:P]
        i_refs = refs[P:2 * P]
        o_refs = refs[2 * P:3 * P]
        sc = refs[3 * P:]
        gsem, wsem = sc[-2], sc[-1]
        base = _wid() * span

        def body(g, carry):
            for u in range(K):
                off = base + (g * K + u) * CH
                for p in range(P):
                    iv = sc[2 * p].at[u]
                    rv = sc[2 * p + 1].at[u]
                    dst = o_refs[p].at[pl.ds(off, CH)]

                    @pl.when(g > 0)
                    def _():
                        pltpu.make_async_copy(rv, dst, wsem).wait()

                    pltpu.sync_copy(i_refs[p].at[pl.ds(off, CH)], iv)
                    pltpu.async_copy(t_refs[p].at[iv], rv, gsem)
            for u in range(K):
                off = base + (g * K + u) * CH
                for p in range(P):
                    iv = sc[2 * p].at[u]
                    rv = sc[2 * p + 1].at[u]
                    pltpu.make_async_copy(t_refs[p].at[iv], rv, gsem).wait()
                    pltpu.async_copy(rv, o_refs[p].at[pl.ds(off, CH)], wsem)
            return carry

        lax.fori_loop(0, ng, body, 0)
        for u in range(K):
            off = base + (nch - K + u) * CH
            for p in range(P):
                rv = sc[2 * p + 1].at[u]
                pltpu.make_async_copy(rv, o_refs[p].at[pl.ds(off, CH)],
                                      wsem).wait()

    return k(*tables, *idxs)


def _sc_gather_combine(pairs, i_idx, j_idx):
    P = len(pairs)
    B = i_idx.shape[0]
    span = B // NW
    assert B % NW == 0 and span % CH == 0
    nch = span // CH
    K = _pipe_depth(nch, 2 * P)
    ng = nch // K
    out_type = tuple(jax.ShapeDtypeStruct((B, t.shape[1]), F32)
                     for t, _, _ in pairs)
    scratch = [pltpu.VMEM((K, CH), I32), pltpu.VMEM((K, CH), I32)]
    for t, _, _ in pairs:
        scratch += [pltpu.VMEM((K, CH, t.shape[1]), F32),
                    pltpu.VMEM((K, CH, t.shape[1]), F32)]
    scratch += [pltpu.SemaphoreType.DMA, pltpu.SemaphoreType.DMA]

    @functools.partial(pl.kernel, out_type=out_type, mesh=_mesh(),
                       scratch_types=scratch)
    def k(*refs):
        t_refs = refs[:2 * P]
        i_ref, j_ref = refs[2 * P], refs[2 * P + 1]
        o_refs = refs[2 * P + 2:3 * P + 2]
        ivb, jvb = refs[3 * P + 2], refs[3 * P + 3]
        bufs = refs[3 * P + 4:5 * P + 4]
        gsem, wsem = refs[-2], refs[-1]
        base = _wid() * span

        def body(g, carry):
            for u in range(K):
                off = base + (g * K + u) * CH
                iv, jv = ivb.at[u], jvb.at[u]

                for p in range(P):
                    r1 = bufs[2 * p].at[u]

                    @pl.when(g > 0)
                    def _():
                        pltpu.make_async_copy(r1, o_refs[p].at[pl.ds(off, CH)],
                                              wsem).wait()

                pltpu.sync_copy(i_ref.at[pl.ds(off, CH)], iv)
                pltpu.sync_copy(j_ref.at[pl.ds(off, CH)], jv)
                for p in range(P):
                    pltpu.async_copy(t_refs[2 * p].at[iv], bufs[2 * p].at[u],
                                     gsem)
                    pltpu.async_copy(t_refs[2 * p + 1].at[jv],
                                     bufs[2 * p + 1].at[u], gsem)
            for u in range(K):
                off = base + (g * K + u) * CH
                iv, jv = ivb.at[u], jvb.at[u]
                for p in range(P):
                    d = pairs[p][0].shape[1]
                    mode = pairs[p][2]
                    r1, r2 = bufs[2 * p].at[u], bufs[2 * p + 1].at[u]
                    pltpu.make_async_copy(t_refs[2 * p].at[iv], r1, gsem).wait()
                    pltpu.make_async_copy(t_refs[2 * p + 1].at[jv], r2,
                                          gsem).wait()
                    if mode == "diff16":
                        for r in range(CH):
                            r1[r, pl.ds(0, 16)] = (r1[r, pl.ds(0, 16)]
                                                   - r2[r, pl.ds(0, 16)])
                    else:
                        def rowadd(r, c2):
                            for s in range(d // 16):
                                sl = pl.ds(s * 16, 16)
                                r1[r, sl] = r1[r, sl] + r2[r, sl]
                            return c2

                        lax.fori_loop(0, CH, rowadd, 0)
                    pltpu.async_copy(r1, o_refs[p].at[pl.ds(off, CH)], wsem)
            return carry

        lax.fori_loop(0, ng, body, 0)
        for u in range(K):
            off = base + (nch - K + u) * CH
            for p in range(P):
                pltpu.make_async_copy(bufs[2 * p].at[u],
                                      o_refs[p].at[pl.ds(off, CH)], wsem).wait()

    tabs = []
    for t_i, t_j, _ in pairs:
        tabs += [t_i, t_j]
    return k(*tabs, i_idx, j_idx)


def _scatter_epilogue(sh, o_refs, cid, sid, rpt, rlast, P):
    @pl.when(sid < NS - 1)
    def _():
        for p in range(P):
            pltpu.sync_copy(sh[p].at[pl.ds(sid * rpt, rpt)],
                            o_refs[p].at[cid, pl.ds(sid * rpt, rpt)])

    @pl.when(sid == NS - 1)
    def _():
        for p in range(P):
            pltpu.sync_copy(sh[p].at[pl.ds((NS - 1) * rpt, rlast)],
                            o_refs[p].at[cid, pl.ds((NS - 1) * rpt, rlast)])


def _sc_scatter(vals, idx, n, zeros, ones_width=None):
    synth = ones_width is not None
    P = 1 if synth else len(vals)
    widths = [ones_width] if synth else [v.shape[1] for v in vals]
    e = idx.shape[0]
    spc = e // NC
    sps = spc // NS
    assert e % (NC * NS) == 0 and sps % CH == 0
    nch = sps // CH
    rpt = ((n // NS) // 8) * 8
    rlast = n - (NS - 1) * rpt
    out_type = tuple(jax.ShapeDtypeStruct((NC, n, d), F32) for d in widths)
    n_in = 0 if synth else P
    K = _pipe_depth(nch, P)
    ng = nch // K
    scratch = [pltpu.VMEM((K, CH), I32)]
    for d in widths:
        scratch.append(pltpu.VMEM((1 if synth else K, CH, d), F32))
    for d in widths:
        scratch.append(pltpu.VMEM_SHARED((n, d), F32))
    scratch += [pltpu.SemaphoreType.DMA, pltpu.SemaphoreType.DMA]

    @functools.partial(pl.kernel, out_type=out_type, mesh=_mesh(),
                       scratch_types=scratch)
    def k(*refs):
        v_refs = refs[:n_in]
        i_ref = refs[n_in]
        z_refs = refs[n_in + 1:n_in + 1 + P]
        o_refs = refs[n_in + 1 + P:n_in + 1 + 2 * P]
        ivb = refs[n_in + 1 + 2 * P]
        vb = refs[n_in + 2 + 2 * P:n_in + 2 + 3 * P]
        sh = refs[n_in + 2 + 3 * P:n_in + 2 + 4 * P]
        vsem, wsem = refs[-2], refs[-1]
        cid = lax.axis_index("c")
        sid = lax.axis_index("s")

        @pl.when(sid == 0)
        def _():
            for p in range(P):
                pltpu.sync_copy(z_refs[p], sh[p])

        if synth:
            def fill(r, carry):
                for s in range(widths[0] // 16):
                    vb[0][0, r, pl.ds(s * 16, 16)] = jnp.ones((16,), F32)
                return carry

            lax.fori_loop(0, CH, fill, 0)

        plsc.subcore_barrier()
        base = cid * spc + sid * sps

        def body(g, carry):
            for u in range(K):
                off = base + (g * K + u) * CH
                iv = ivb.at[u]
                for p in range(P):
                    bv = vb[p].at[0 if synth else u]

                    @pl.when(g > 0)
                    def _():
                        pltpu.make_async_copy(bv, sh[p].at[iv], wsem).wait()

                pltpu.sync_copy(i_ref.at[pl.ds(off, CH)], iv)
                if not synth:
                    for p in range(P):
                        pltpu.async_copy(v_refs[p].at[pl.ds(off, CH)],
                                         vb[p].at[u], vsem)
            for u in range(K):
                off = base + (g * K + u) * CH
                iv = ivb.at[u]
                for p in range(P):
                    bv = vb[p].at[0 if synth else u]
                    if not synth:
                        pltpu.make_async_copy(v_refs[p].at[pl.ds(off, CH)],
                                              bv, vsem).wait()
                    pltpu.async_copy(bv, sh[p].at[iv], wsem, add=True)
            return carry

        lax.fori_loop(0, ng, body, 0)
        for u in range(K):
            iv = ivb.at[u]
            for p in range(P):
                bv = vb[p].at[0 if synth else u]
                pltpu.make_async_copy(bv, sh[p].at[iv], wsem).wait()
        plsc.subcore_barrier()
        _scatter_epilogue(sh, o_refs, cid, sid, rpt, rlast, P)

    if synth:
        return k(idx, *zeros)
    return k(*vals, idx, *zeros)


def _sc_gather_scatter(table, gidx, sidx, n, zeros):
    d = table.shape[1]
    e = gidx.shape[0]
    spc = e // NC
    sps = spc // NS
    assert e % (NC * NS) == 0 and sps % CH == 0
    nch = sps // CH
    rpt = ((n // NS) // 8) * 8
    rlast = n - (NS - 1) * rpt

    K = _pipe_depth(nch, 1)
    ng = nch // K

    @functools.partial(
        pl.kernel, out_type=(jax.ShapeDtypeStruct((NC, n, d), F32),),
        mesh=_mesh(),
        scratch_types=[pltpu.VMEM((K, CH), I32), pltpu.VMEM((K, CH), I32),
                       pltpu.VMEM((K, CH, d), F32),
                       pltpu.VMEM_SHARED((n, d), F32),
                       pltpu.SemaphoreType.DMA, pltpu.SemaphoreType.DMA])
    def k(t_ref, g_ref, s_ref, z_ref, o_ref, gvb, svb, vb, sh, gsem, wsem):
        cid = lax.axis_index("c")
        sid = lax.axis_index("s")

        @pl.when(sid == 0)
        def _():
            pltpu.sync_copy(z_ref, sh)

        plsc.subcore_barrier()
        base = cid * spc + sid * sps

        def body(g, carry):
            for u in range(K):
                off = base + (g * K + u) * CH
                gv, sv, bv = gvb.at[u], svb.at[u], vb.at[u]

                @pl.when(g > 0)
                def _():
                    pltpu.make_async_copy(bv, sh.at[sv], wsem).wait()

                pltpu.sync_copy(g_ref.at[pl.ds(off, CH)], gv)
                pltpu.sync_copy(s_ref.at[pl.ds(off, CH)], sv)
                pltpu.async_copy(t_ref.at[gv], bv, gsem)
            for u in range(K):
                gv, sv, bv = gvb.at[u], svb.at[u], vb.at[u]
                pltpu.make_async_copy(t_ref.at[gv], bv, gsem).wait()
                pltpu.async_copy(bv, sh.at[sv], wsem, add=True)
            return carry

        lax.fori_loop(0, ng, body, 0)
        for u in range(K):
            pltpu.make_async_copy(vb.at[u], sh.at[svb.at[u]], wsem).wait()
        plsc.subcore_barrier()
        _scatter_epilogue([sh], [o_ref], cid, sid, rpt, rlast, 1)

    return k(table, gidx, sidx, zeros)[0]


def _sc_win(idp, n, padn):
    np2 = ((n + 127) // 128) * 128
    spw = padn // NW
    ng = spw // 16

    @functools.partial(
        pl.kernel, out_type=jax.ShapeDtypeStruct((NW * np2,), I32),
        mesh=_mesh(),
        scratch_types=[pltpu.VMEM((padn,), I32), pltpu.VMEM((np2 + 16,), I32)])
    def k(id_ref, o_ref, idv, acc):
        w = _wid()
        pltpu.sync_copy(id_ref, idv)

        def z(c, carry):
            acc[pl.ds(c * 16, 16)] = jnp.full((16,), -1, I32)
            return carry

        lax.fori_loop(0, (np2 + 16) // 16, z, 0)
        k0 = w * spw
        iot = lax.iota(I32, 16)

        def outer(g, carry):
            kbase = k0 + g * 16
            idvec = idv[pl.ds(kbase, 16)]
            for l in range(16):
                kk = kbase + l

                @pl.when(kk < n)
                def _():
                    t = idvec[l]
                    cur = acc[pl.ds(t, 16)]
                    acc[pl.ds(t, 16)] = jnp.where(iot == 0, kk, cur)
            return carry

        lax.fori_loop(0, ng, outer, 0)
        pltpu.sync_copy(acc.at[pl.ds(0, np2)], o_ref.at[pl.ds(w * np2, np2)])

    return k(idp).reshape(NW, np2)


_BN = 1000


def _row_spec(d):
    return pl.BlockSpec((_BN, d), lambda i: (i, 0))


def _full_spec(r, c):
    return pl.BlockSpec((r, c), lambda i: (0, 0))


def _tc_prep(x_list, posp, wa, wb, c3):
    n, lat = x_list[0].shape
    nx = len(x_list)

    def body(*refs):
        xs = refs[:nx]
        pp, wa_r, wb_r, c3_r, a_r, b_r = refs[nx:]
        x = xs[0][...]
        for r in xs[1:]:
            x = x + r[...]
        pc = jnp.dot(pp[...], c3_r[...], preferred_element_type=F32)
        a_r[...] = jnp.dot(x, wa_r[...], preferred_element_type=F32) + pc
        b_r[...] = jnp.dot(x, wb_r[...], preferred_element_type=F32) - pc

    return pl.pallas_call(
        body, grid=(n // _BN,),
        in_specs=[_row_spec(lat)] * nx + [_row_spec(LAT), _full_spec(lat, lat),
                                          _full_spec(lat, lat),
                                          _full_spec(LAT, lat)],
        out_specs=[_row_spec(lat)] * 2,
        out_shape=[jax.ShapeDtypeStruct((n, lat), F32)] * 2,
    )(*x_list, posp, wa, wb, c3)


def _ln(y, g, b):
    mu = jnp.mean(y, axis=-1, keepdims=True)
    yc = y - mu
    var = jnp.mean(yc * yc, axis=-1, keepdims=True)
    return yc * lax.rsqrt(var + 1e-5) * g + b


def _tc_edge(gab, dd, c4, b0, w1, b1, g, b):
    e, lat = gab.shape

    def body(gab_r, dd_r, c4_r, b0_r, w1_r, b1_r, g_r, b_r, o_r):
        d = dd_r[:, :16]
        nrm = jnp.sqrt(jnp.sum(d * d, axis=-1, keepdims=True) + 1e-12)
        y = gab_r[...] + nrm * c4_r[...] + b0_r[...]
        y = jnp.maximum(y, 0.0)
        y = jnp.dot(y, w1_r[...], preferred_element_type=F32) + b1_r[...]
        o_r[...] = _ln(y, g_r[...], b_r[...])

    return pl.pallas_call(
        body, grid=(e // _BN,),
        in_specs=[_row_spec(lat)] * 2
        + [_full_spec(1, lat), _full_spec(1, lat), _full_spec(lat, lat),
           _full_spec(1, lat), _full_spec(1, lat), _full_spec(1, lat)],
        out_specs=_row_spec(lat),
        out_shape=jax.ShapeDtypeStruct((e, lat), F32))(
            gab, dd, c4, b0, w1, b1, g, b)


def _tc_node(x_list, a0, a1, wx, wa, b0, w1, b1, g, b, extra=None):
    n, lat = x_list[0].shape
    nx = len(x_list)
    ins = list(x_list) + [a0, a1, wx, wa, b0, w1, b1, g, b]
    if extra is not None:
        ins.append(extra)
    ne = extra is not None

    def body(*refs):
        xs = refs[:nx]
        a0_r, a1_r, wx_r, wa_r, b0_r, w1_r, b1_r, g_r, b_r = refs[nx:nx + 9]
        o_r = refs[-1]
        x = xs[0][...]
        for r in xs[1:]:
            x = x + r[...]
        ag = a0_r[...] + a1_r[...]
        y = (jnp.dot(x, wx_r[...], preferred_element_type=F32)
             + jnp.dot(ag, wa_r[...], preferred_element_type=F32) + b0_r[...])
        y = jnp.maximum(y, 0.0)
        y = jnp.dot(y, w1_r[...], preferred_element_type=F32) + b1_r[...]
        y = _ln(y, g_r[...], b_r[...]) + x
        if ne:
            y = y + refs[nx + 9][...]
        o_r[...] = y

    return pl.pallas_call(
        body, grid=(n // _BN,),
        in_specs=[_row_spec(lat)] * nx
        + [_row_spec(lat), _row_spec(lat), _full_spec(lat, lat),
           _full_spec(lat, lat), _full_spec(1, lat), _full_spec(lat, lat),
           _full_spec(1, lat), _full_spec(1, lat), _full_spec(1, lat)]
        + ([_row_spec(lat)] if ne else []),
        out_specs=_row_spec(lat),
        out_shape=jax.ShapeDtypeStruct((n, lat), F32))(*ins)


def _tc_combine(parts, mults, post=None):
    nc, n, d = parts.shape

    def body(*refs):
        in_r = refs[0]
        o_r = refs[-1]
        s = in_r[0] + in_r[1]
        if post == "inv":
            s = 1.0 / s
        elif post == "inveps":
            s = 1.0 / (s + 1e-12)
        for m_r in refs[1:-1]:
            s = s * m_r[...]
        o_r[...] = s

    return pl.pallas_call(
        body, grid=(n // _BN,),
        in_specs=[pl.BlockSpec((nc, _BN, d), lambda i: (0, i, 0))]
        + [_row_spec(d)] * len(mults),
        out_specs=_row_spec(d),
        out_shape=jax.ShapeDtypeStruct((n, d), F32))(parts, *mults)


def _tc_mul(a, b, col=None):
    n, d = a.shape
    ins = [a, b] + ([col] if col is not None else [])

    def body(*refs):
        o_r = refs[-1]
        y = refs[0][...] * refs[1][...]
        if col is not None:
            y = y * refs[2][...]
        o_r[...] = y

    return pl.pallas_call(
        body, grid=(n // _BN,),
        in_specs=[_row_spec(d), _row_spec(d)]
        + ([_row_spec(1)] if col is not None else []),
        out_specs=_row_spec(d),
        out_shape=jax.ShapeDtypeStruct((n, d), F32))(*ins)


def _tc_winmax(parts):
    p, n = parts.shape

    def body(in_r, w_r, m_r):
        m = jnp.max(in_r[...], axis=0, keepdims=True)
        w_r[...] = jnp.maximum(m, 0)
        m_r[...] = jnp.where(m >= 0, 1.0, 0.0).astype(F32)

    return pl.pallas_call(
        body, grid=(1,),
        in_specs=[pl.BlockSpec((p, n), lambda i: (0, 0))],
        out_specs=[pl.BlockSpec((1, n), lambda i: (0, 0))] * 2,
        out_shape=[jax.ShapeDtypeStruct((1, n), I32),
                   jax.ShapeDtypeStruct((1, n), F32)])(parts)


def _edge_w(p, lat):
    w0 = p["w0"]
    c3 = jnp.pad(w0[0:3], ((0, LAT - 3), (0, 0)))
    c4 = w0[3:4]
    wa = w0[4:4 + lat]
    wb = w0[4 + lat:4 + 2 * lat]
    return (c3, c4, wa, wb, p["b0"].reshape(1, -1), p["w1"],
            p["b1"].reshape(1, -1), p["ln_g"].reshape(1, -1),
            p["ln_b"].reshape(1, -1))


def _node_w(p, lat):
    return (p["w0"][:lat], p["w0"][lat:], p["b0"].reshape(1, -1), p["w1"],
            p["b1"].reshape(1, -1), p["ln_g"].reshape(1, -1),
            p["ln_b"].reshape(1, -1))


def _gmp(x_list, dd, i_idx, j_idx, posk, pe, pn, n, lat, zz, extra=None):
    c3, c4, wa, wb, b0e, w1e, b1e, ge, be = _edge_w(pe, lat)
    a_t, b_t = _tc_prep(x_list, posk, wa, wb, c3)
    if dd is None:
        gab, dd = _sc_gather_combine(
            [(a_t, b_t, "add"), (posk, posk, "diff16")], i_idx, j_idx)
    else:
        (gab,) = _sc_gather_combine([(a_t, b_t, "add")], i_idx, j_idx)
    emb = _tc_edge(gab, dd, c4, b0e, w1e, b1e, ge, be)
    (agp,) = _sc_scatter([emb], j_idx, n, [zz])
    wx, wac, b0n, w1n, b1n, gn, bn = _node_w(pn, lat)
    out = _tc_node(x_list, agp[0], agp[1], wx, wac, b0n, w1n, b1n, gn, bn,
                   extra=extra)
    return out, dd


def kernel(h, m_ids, m_gs, pos, params):
    n, lat = h.shape
    e = m_gs.shape[-1]
    i0 = m_gs[0, 0]
    j0 = m_gs[0, 1]
    i1 = m_gs[1, 0]
    j1 = m_gs[1, 1]
    ids0 = m_ids[0]
    padn = ((n + NW * CH - 1) // (NW * CH)) * NW * CH

    posp = jnp.pad(pos, ((0, 0), (0, LAT - pos.shape[1])))
    zz128 = jnp.zeros((n, lat), F32)

    h1, d0 = _gmp([h], None, i0, j0, posp, params["down0"]["mlp_edge"],
                  params["down0"]["mlp_node"], n, lat, zz128)

    (degp,) = _sc_scatter(None, i0, n, [zz128], ones_width=lat)
    nw128 = _tc_combine(degp, [], post="inv")
    awp = _sc_gather_scatter(nw128, i0, j0, n, zz128)
    iaw128 = _tc_combine(awp, [], post="inveps")

    hw1 = _tc_mul(h1, nw128)
    posw = _tc_mul(posp, nw128)
    shp = _sc_gather_scatter(hw1, i0, j0, n, zz128)
    spp = _sc_gather_scatter(posw, i0, j0, n, zz128)
    h2f = _tc_combine(shp, [iaw128])
    p2f = _tc_combine(spp, [iaw128])
    idp = jnp.pad(ids0, (0, padn - n))
    h2p, p2p = _sc_gather([h2f, p2f], [idp, idp])
    h2 = h2p[:n]
    p2 = p2p[:n]

    h3, _ = _gmp([h2], None, i1, j1, p2, params["bottom"]["mlp_edge"],
                 params["bottom"]["mlp_node"], n, lat, zz128)

    winp = _sc_win(idp, n, padn)
    winc, maskf = _tc_winmax(winp)
    wpad = jnp.pad(winc.reshape(-1)[:n], (0, padn - n))
    (h4p,) = _sc_gather([h3], [wpad])

    hw4 = _tc_mul(h4p[:n], iaw128, col=maskf.reshape(-1)[:n].reshape(n, 1))
    up_ = _sc_gather_scatter(hw4, j0, i0, n, zz128)
    hu = _tc_combine(up_, [nw128])

    out, _ = _gmp([hu], d0, i0, j0, posp, params["up0"]["mlp_edge"],
                  params["up0"]["mlp_node"], n, lat, zz128, extra=h1)
    return out


# speedup vs baseline: 4.2831x; 1.0671x over previous
import functools

import jax
import jax.numpy as jnp
from jax import lax
from jax.experimental import pallas as pl
from jax.experimental.pallas import tpu as pltpu
from jax.experimental.pallas import tpu_sc as plsc

F32 = jnp.float32
I32 = jnp.int32
NC, NS = 2, 16
NW = NC * NS
CH = 40
LAT = 128


def _wid():
    return lax.axis_index("s") * NC + lax.axis_index("c")


def _mesh():
    return plsc.VectorSubcoreMesh(core_axis_name="c", subcore_axis_name="s")


def _pipe_depth(nch, p):
    for kd in (5, 4, 2):
        if nch % kd == 0 and kd * p <= 10:
            return kd
    return 1


def _sc_gather(tables, idxs):
    P = len(tables)
    B = idxs[0].shape[0]
    span = B // NW
    assert B % NW == 0 and span % CH == 0
    nch = span // CH
    K = _pipe_depth(nch, P)
    ng = nch // K
    out_type = tuple(jax.ShapeDtypeStruct((B, t.shape[1]), t.dtype)
                     for t in tables)
    scratch = []
    for t in tables:
        scratch += [pltpu.VMEM((K, CH), I32),
                    pltpu.VMEM((K, CH, t.shape[1]), t.dtype)]
    scratch += [pltpu.SemaphoreType.DMA, pltpu.SemaphoreType.DMA]

    @functools.partial(pl.kernel, out_type=out_type, mesh=_mesh(),
                       scratch_types=scratch)
    def k(*refs):
        t_refs = refs[:P]
        i_refs = refs[P:2 * P]
        o_refs = refs[2 * P:3 * P]
        sc = refs[3 * P:]
        gsem, wsem = sc[-2], sc[-1]
        base = _wid() * span

        def body(g, carry):
            for u in range(K):
                off = base + (g * K + u) * CH
                for p in range(P):
                    iv = sc[2 * p].at[u]
                    rv = sc[2 * p + 1].at[u]
                    dst = o_refs[p].at[pl.ds(off, CH)]

                    @pl.when(g > 0)
                    def _():
                        pltpu.make_async_copy(rv, dst, wsem).wait()

                    pltpu.sync_copy(i_refs[p].at[pl.ds(off, CH)], iv)
                    pltpu.async_copy(t_refs[p].at[iv], rv, gsem)
            for u in range(K):
                off = base + (g * K + u) * CH
                for p in range(P):
                    iv = sc[2 * p].at[u]
                    rv = sc[2 * p + 1].at[u]
                    pltpu.make_async_copy(t_refs[p].at[iv], rv, gsem).wait()
                    pltpu.async_copy(rv, o_refs[p].at[pl.ds(off, CH)], wsem)
            return carry

        lax.fori_loop(0, ng, body, 0)
        for u in range(K):
            off = base + (nch - K + u) * CH
            for p in range(P):
                rv = sc[2 * p + 1].at[u]
                pltpu.make_async_copy(rv, o_refs[p].at[pl.ds(off, CH)],
                                      wsem).wait()

    return k(*tables, *idxs)


def _sc_gather_combine(pairs, i_idx, j_idx):
    P = len(pairs)
    B = i_idx.shape[0]
    span = B // NW
    assert B % NW == 0 and span % CH == 0
    nch = span // CH
    K = _pipe_depth(nch, 2 * P)
    ng = nch // K
    out_type = tuple(jax.ShapeDtypeStruct((B, t.shape[1]), F32)
                     for t, _, _ in pairs)
    scratch = [pltpu.VMEM((K, CH), I32), pltpu.VMEM((K, CH), I32)]
    for t, _, _ in pairs:
        scratch += [pltpu.VMEM((K, CH, t.shape[1]), F32),
                    pltpu.VMEM((K, CH, t.shape[1]), F32)]
    scratch += [pltpu.SemaphoreType.DMA, pltpu.SemaphoreType.DMA]

    @functools.partial(pl.kernel, out_type=out_type, mesh=_mesh(),
                       scratch_types=scratch)
    def k(*refs):
        t_refs = refs[:2 * P]
        i_ref, j_ref = refs[2 * P], refs[2 * P + 1]
        o_refs = refs[2 * P + 2:3 * P + 2]
        ivb, jvb = refs[3 * P + 2], refs[3 * P + 3]
        bufs = refs[3 * P + 4:5 * P + 4]
        gsem, wsem = refs[-2], refs[-1]
        base = _wid() * span

        def body(g, carry):
            for u in range(K):
                off = base + (g * K + u) * CH
                iv, jv = ivb.at[u], jvb.at[u]

                for p in range(P):
                    r1 = bufs[2 * p].at[u]

                    @pl.when(g > 0)
                    def _():
                        pltpu.make_async_copy(r1, o_refs[p].at[pl.ds(off, CH)],
                                              wsem).wait()

                pltpu.sync_copy(i_ref.at[pl.ds(off, CH)], iv)
                pltpu.sync_copy(j_ref.at[pl.ds(off, CH)], jv)
                for p in range(P):
                    pltpu.async_copy(t_refs[2 * p].at[iv], bufs[2 * p].at[u],
                                     gsem)
                    pltpu.async_copy(t_refs[2 * p + 1].at[jv],
                                     bufs[2 * p + 1].at[u], gsem)
            for u in range(K):
                off = base + (g * K + u) * CH
                iv, jv = ivb.at[u], jvb.at[u]
                for p in range(P):
                    d = pairs[p][0].shape[1]
                    mode = pairs[p][2]
                    r1, r2 = bufs[2 * p].at[u], bufs[2 * p + 1].at[u]
                    pltpu.make_async_copy(t_refs[2 * p].at[iv], r1, gsem).wait()
                    pltpu.make_async_copy(t_refs[2 * p + 1].at[jv], r2,
                                          gsem).wait()
                    if mode == "diff16":
                        for r in range(CH):
                            r1[r, pl.ds(0, 16)] = (r1[r, pl.ds(0, 16)]
                                                   - r2[r, pl.ds(0, 16)])
                    else:
                        def rowadd(r, c2):
                            for s in range(d // 16):
                                sl = pl.ds(s * 16, 16)
                                r1[r, sl] = r1[r, sl] + r2[r, sl]
                            return c2

                        lax.fori_loop(0, CH, rowadd, 0)
                    pltpu.async_copy(r1, o_refs[p].at[pl.ds(off, CH)], wsem)
            return carry

        lax.fori_loop(0, ng, body, 0)
        for u in range(K):
            off = base + (nch - K + u) * CH
            for p in range(P):
                pltpu.make_async_copy(bufs[2 * p].at[u],
                                      o_refs[p].at[pl.ds(off, CH)], wsem).wait()

    tabs = []
    for t_i, t_j, _ in pairs:
        tabs += [t_i, t_j]
    return k(*tabs, i_idx, j_idx)


def _scatter_epilogue(sh, o_refs, cid, sid, rpt, rlast, P):
    @pl.when(sid < NS - 1)
    def _():
        for p in range(P):
            pltpu.sync_copy(sh[p].at[pl.ds(sid * rpt, rpt)],
                            o_refs[p].at[cid, pl.ds(sid * rpt, rpt)])

    @pl.when(sid == NS - 1)
    def _():
        for p in range(P):
            pltpu.sync_copy(sh[p].at[pl.ds((NS - 1) * rpt, rlast)],
                            o_refs[p].at[cid, pl.ds((NS - 1) * rpt, rlast)])


def _sc_scatter(vals, idx, n, zeros, ones_width=None):
    synth = ones_width is not None
    P = 1 if synth else len(vals)
    widths = [ones_width] if synth else [v.shape[1] for v in vals]
    e = idx.shape[0]
    spc = e // NC
    sps = spc // NS
    assert e % (NC * NS) == 0 and sps % CH == 0
    nch = sps // CH
    rpt = ((n // NS) // 8) * 8
    rlast = n - (NS - 1) * rpt
    out_type = tuple(jax.ShapeDtypeStruct((NC, n, d), F32) for d in widths)
    n_in = 0 if synth else P
    K = _pipe_depth(nch, P)
    ng = nch // K
    scratch = [pltpu.VMEM((K, CH), I32)]
    for d in widths:
        scratch.append(pltpu.VMEM((1 if synth else K, CH, d), F32))
    for d in widths:
        scratch.append(pltpu.VMEM_SHARED((n, d), F32))
    scratch += [pltpu.SemaphoreType.DMA, pltpu.SemaphoreType.DMA]

    @functools.partial(pl.kernel, out_type=out_type, mesh=_mesh(),
                       scratch_types=scratch)
    def k(*refs):
        v_refs = refs[:n_in]
        i_ref = refs[n_in]
        z_refs = refs[n_in + 1:n_in + 1 + P]
        o_refs = refs[n_in + 1 + P:n_in + 1 + 2 * P]
        ivb = refs[n_in + 1 + 2 * P]
        vb = refs[n_in + 2 + 2 * P:n_in + 2 + 3 * P]
        sh = refs[n_in + 2 + 3 * P:n_in + 2 + 4 * P]
        vsem, wsem = refs[-2], refs[-1]
        cid = lax.axis_index("c")
        sid = lax.axis_index("s")

        @pl.when(sid == 0)
        def _():
            for p in range(P):
                pltpu.sync_copy(z_refs[p], sh[p])

        if synth:
            def fill(r, carry):
                for s in range(widths[0] // 16):
                    vb[0][0, r, pl.ds(s * 16, 16)] = jnp.ones((16,), F32)
                return carry

            lax.fori_loop(0, CH, fill, 0)

        plsc.subcore_barrier()
        base = cid * spc + sid * sps

        def body(g, carry):
            for u in range(K):
                off = base + (g * K + u) * CH
                iv = ivb.at[u]
                for p in range(P):
                    bv = vb[p].at[0 if synth else u]

                    @pl.when(g > 0)
                    def _():
                        pltpu.make_async_copy(bv, sh[p].at[iv], wsem).wait()

                pltpu.sync_copy(i_ref.at[pl.ds(off, CH)], iv)
                if not synth:
                    for p in range(P):
                        pltpu.async_copy(v_refs[p].at[pl.ds(off, CH)],
                                         vb[p].at[u], vsem)
            for u in range(K):
                off = base + (g * K + u) * CH
                iv = ivb.at[u]
                for p in range(P):
                    bv = vb[p].at[0 if synth else u]
                    if not synth:
                        pltpu.make_async_copy(v_refs[p].at[pl.ds(off, CH)],
                                              bv, vsem).wait()
                    pltpu.async_copy(bv, sh[p].at[iv], wsem, add=True)
            return carry

        lax.fori_loop(0, ng, body, 0)
        for u in range(K):
            iv = ivb.at[u]
            for p in range(P):
                bv = vb[p].at[0 if synth else u]
                pltpu.make_async_copy(bv, sh[p].at[iv], wsem).wait()
        plsc.subcore_barrier()
        _scatter_epilogue(sh, o_refs, cid, sid, rpt, rlast, P)

    if synth:
        return k(idx, *zeros)
    return k(*vals, idx, *zeros)


def _sc_gather_scatter(table, gidx, sidx, n, zeros):
    d = table.shape[1]
    e = gidx.shape[0]
    spc = e // NC
    sps = spc // NS
    assert e % (NC * NS) == 0 and sps % CH == 0
    nch = sps // CH
    rpt = ((n // NS) // 8) * 8
    rlast = n - (NS - 1) * rpt

    K = _pipe_depth(nch, 1)
    ng = nch // K

    @functools.partial(
        pl.kernel, out_type=(jax.ShapeDtypeStruct((NC, n, d), F32),),
        mesh=_mesh(),
        scratch_types=[pltpu.VMEM((K, CH), I32), pltpu.VMEM((K, CH), I32),
                       pltpu.VMEM((K, CH, d), F32),
                       pltpu.VMEM_SHARED((n, d), F32),
                       pltpu.SemaphoreType.DMA, pltpu.SemaphoreType.DMA])
    def k(t_ref, g_ref, s_ref, z_ref, o_ref, gvb, svb, vb, sh, gsem, wsem):
        cid = lax.axis_index("c")
        sid = lax.axis_index("s")

        @pl.when(sid == 0)
        def _():
            pltpu.sync_copy(z_ref, sh)

        plsc.subcore_barrier()
        base = cid * spc + sid * sps

        def body(g, carry):
            for u in range(K):
                off = base + (g * K + u) * CH
                gv, sv, bv = gvb.at[u], svb.at[u], vb.at[u]

                @pl.when(g > 0)
                def _():
                    pltpu.make_async_copy(bv, sh.at[sv], wsem).wait()

                pltpu.sync_copy(g_ref.at[pl.ds(off, CH)], gv)
                pltpu.sync_copy(s_ref.at[pl.ds(off, CH)], sv)
                pltpu.async_copy(t_ref.at[gv], bv, gsem)
            for u in range(K):
                gv, sv, bv = gvb.at[u], svb.at[u], vb.at[u]
                pltpu.make_async_copy(t_ref.at[gv], bv, gsem).wait()
                pltpu.async_copy(bv, sh.at[sv], wsem, add=True)
            return carry

        lax.fori_loop(0, ng, body, 0)
        for u in range(K):
            pltpu.make_async_copy(vb.at[u], sh.at[svb.at[u]], wsem).wait()
        plsc.subcore_barrier()
        _scatter_epilogue([sh], [o_ref], cid, sid, rpt, rlast, 1)

    return k(table, gidx, sidx, zeros)[0]


def _sc_win(idp, n, padn):
    np2 = ((n + 127) // 128) * 128
    spw = padn // NW
    ng = spw // 16

    @functools.partial(
        pl.kernel, out_type=jax.ShapeDtypeStruct((NW * np2,), I32),
        mesh=_mesh(),
        scratch_types=[pltpu.VMEM((padn,), I32), pltpu.VMEM((np2 + 16,), I32)])
    def k(id_ref, o_ref, idv, acc):
        w = _wid()
        pltpu.sync_copy(id_ref, idv)

        def z(c, carry):
            acc[pl.ds(c * 16, 16)] = jnp.full((16,), -1, I32)
            return carry

        lax.fori_loop(0, (np2 + 16) // 16, z, 0)
        k0 = w * spw
        iot = lax.iota(I32, 16)

        def outer(g, carry):
            kbase = k0 + g * 16
            idvec = idv[pl.ds(kbase, 16)]
            for l in range(16):
                kk = kbase + l

                @pl.when(kk < n)
                def _():
                    t = idvec[l]
                    cur = acc[pl.ds(t, 16)]
                    acc[pl.ds(t, 16)] = jnp.where(iot == 0, kk, cur)
            return carry

        lax.fori_loop(0, ng, outer, 0)
        pltpu.sync_copy(acc.at[pl.ds(0, np2)], o_ref.at[pl.ds(w * np2, np2)])

    return k(idp).reshape(NW, np2)


_BN = 1000


def _row_spec(d):
    return pl.BlockSpec((_BN, d), lambda i: (i, 0))


def _full_spec(r, c):
    return pl.BlockSpec((r, c), lambda i: (0, 0))


def _tc_prep(x_list, posp, wa, wb, c3):
    n, lat = x_list[0].shape
    nx = len(x_list)

    def body(*refs):
        xs = refs[:nx]
        pp, wa_r, wb_r, c3_r, a_r, b_r = refs[nx:]
        x = xs[0][...]
        for r in xs[1:]:
            x = x + r[...]
        pc = jnp.dot(pp[...], c3_r[...], preferred_element_type=F32)
        a_r[...] = jnp.dot(x, wa_r[...], preferred_element_type=F32) + pc
        b_r[...] = jnp.dot(x, wb_r[...], preferred_element_type=F32) - pc

    return pl.pallas_call(
        body, grid=(n // _BN,),
        in_specs=[_row_spec(lat)] * nx + [_row_spec(LAT), _full_spec(lat, lat),
                                          _full_spec(lat, lat),
                                          _full_spec(LAT, lat)],
        out_specs=[_row_spec(lat)] * 2,
        out_shape=[jax.ShapeDtypeStruct((n, lat), F32)] * 2,
    )(*x_list, posp, wa, wb, c3)


def _ln(y, g, b):
    mu = jnp.mean(y, axis=-1, keepdims=True)
    yc = y - mu
    var = jnp.mean(yc * yc, axis=-1, keepdims=True)
    return yc * lax.rsqrt(var + 1e-5) * g + b


def _tc_edge(gab, dd, c4, b0, w1, b1, g, b):
    e, lat = gab.shape

    def body(gab_r, dd_r, c4_r, b0_r, w1_r, b1_r, g_r, b_r, o_r):
        d = dd_r[:, :16]
        nrm = jnp.sqrt(jnp.sum(d * d, axis=-1, keepdims=True) + 1e-12)
        y = gab_r[...] + nrm * c4_r[...] + b0_r[...]
        y = jnp.maximum(y, 0.0)
        y = jnp.dot(y, w1_r[...], preferred_element_type=F32) + b1_r[...]
        o_r[...] = _ln(y, g_r[...], b_r[...])

    return pl.pallas_call(
        body, grid=(e // _BN,),
        in_specs=[_row_spec(lat)] * 2
        + [_full_spec(1, lat), _full_spec(1, lat), _full_spec(lat, lat),
           _full_spec(1, lat), _full_spec(1, lat), _full_spec(1, lat)],
        out_specs=_row_spec(lat),
        out_shape=jax.ShapeDtypeStruct((e, lat), F32))(
            gab, dd, c4, b0, w1, b1, g, b)


def _tc_node(x_list, a0, a1, wx, wa, b0, w1, b1, g, b, extra=None):
    n, lat = x_list[0].shape
    nx = len(x_list)
    ins = list(x_list) + [a0, a1, wx, wa, b0, w1, b1, g, b]
    if extra is not None:
        ins.append(extra)
    ne = extra is not None

    def body(*refs):
        xs = refs[:nx]
        a0_r, a1_r, wx_r, wa_r, b0_r, w1_r, b1_r, g_r, b_r = refs[nx:nx + 9]
        o_r = refs[-1]
        x = xs[0][...]
        for r in xs[1:]:
            x = x + r[...]
        ag = a0_r[...] + a1_r[...]
        y = (jnp.dot(x, wx_r[...], preferred_element_type=F32)
             + jnp.dot(ag, wa_r[...], preferred_element_type=F32) + b0_r[...])
        y = jnp.maximum(y, 0.0)
        y = jnp.dot(y, w1_r[...], preferred_element_type=F32) + b1_r[...]
        y = _ln(y, g_r[...], b_r[...]) + x
        if ne:
            y = y + refs[nx + 9][...]
        o_r[...] = y

    return pl.pallas_call(
        body, grid=(n // _BN,),
        in_specs=[_row_spec(lat)] * nx
        + [_row_spec(lat), _row_spec(lat), _full_spec(lat, lat),
           _full_spec(lat, lat), _full_spec(1, lat), _full_spec(lat, lat),
           _full_spec(1, lat), _full_spec(1, lat), _full_spec(1, lat)]
        + ([_row_spec(lat)] if ne else []),
        out_specs=_row_spec(lat),
        out_shape=jax.ShapeDtypeStruct((n, lat), F32))(*ins)


def _tc_combine(parts, mults, post=None):
    nc, n, d = parts.shape

    def body(*refs):
        in_r = refs[0]
        o_r = refs[-1]
        s = in_r[0] + in_r[1]
        if post == "inv":
            s = 1.0 / s
        elif post == "inveps":
            s = 1.0 / (s + 1e-12)
        for m_r in refs[1:-1]:
            s = s * m_r[...]
        o_r[...] = s

    return pl.pallas_call(
        body, grid=(n // _BN,),
        in_specs=[pl.BlockSpec((nc, _BN, d), lambda i: (0, i, 0))]
        + [_row_spec(d)] * len(mults),
        out_specs=_row_spec(d),
        out_shape=jax.ShapeDtypeStruct((n, d), F32))(parts, *mults)


def _tc_mul(a, b, col=None):
    n, d = a.shape
    ins = [a, b] + ([col] if col is not None else [])

    def body(*refs):
        o_r = refs[-1]
        y = refs[0][...] * refs[1][...]
        if col is not None:
            y = y * refs[2][...]
        o_r[...] = y

    return pl.pallas_call(
        body, grid=(n // _BN,),
        in_specs=[_row_spec(d), _row_spec(d)]
        + ([_row_spec(1)] if col is not None else []),
        out_specs=_row_spec(d),
        out_shape=jax.ShapeDtypeStruct((n, d), F32))(*ins)


def _tc_postable(posp, nw128):
    n, d = posp.shape

    def body(pp_r, nw_r, o_r):
        lane = lax.broadcasted_iota(I32, (_BN, d), 1)
        o_r[...] = (pp_r[...] + jnp.where(lane == 3, 1.0, 0.0)) * nw_r[...]

    return pl.pallas_call(
        body, grid=(n // _BN,),
        in_specs=[_row_spec(d), _row_spec(d)],
        out_specs=_row_spec(d),
        out_shape=jax.ShapeDtypeStruct((n, d), F32))(posp, nw128)


def _tc_unpack_s(parts):
    nc, n, d = parts.shape

    def body(in_r, o1_r, o2_r):
        s = in_r[0] + in_r[1]
        iaw = 1.0 / (s[:, 3:4] + 1e-12)
        o1_r[...] = jnp.broadcast_to(iaw, s.shape)
        lane = lax.broadcasted_iota(I32, s.shape, 1)
        o2_r[...] = jnp.where(lane == 3, 0.0, s * iaw)

    return pl.pallas_call(
        body, grid=(n // _BN,),
        in_specs=[pl.BlockSpec((nc, _BN, d), lambda i: (0, i, 0))],
        out_specs=[_row_spec(d)] * 2,
        out_shape=[jax.ShapeDtypeStruct((n, d), F32)] * 2)(parts)


def _tc_winmax(parts):
    p, n = parts.shape

    def body(in_r, w_r, m_r):
        m = jnp.max(in_r[...], axis=0, keepdims=True)
        w_r[...] = jnp.maximum(m, 0)
        m_r[...] = jnp.where(m >= 0, 1.0, 0.0).astype(F32)

    return pl.pallas_call(
        body, grid=(1,),
        in_specs=[pl.BlockSpec((p, n), lambda i: (0, 0))],
        out_specs=[pl.BlockSpec((1, n), lambda i: (0, 0))] * 2,
        out_shape=[jax.ShapeDtypeStruct((1, n), I32),
                   jax.ShapeDtypeStruct((1, n), F32)])(parts)


def _edge_w(p, lat):
    w0 = p["w0"]
    c3 = jnp.pad(w0[0:3], ((0, LAT - 3), (0, 0)))
    c4 = w0[3:4]
    wa = w0[4:4 + lat]
    wb = w0[4 + lat:4 + 2 * lat]
    return (c3, c4, wa, wb, p["b0"].reshape(1, -1), p["w1"],
            p["b1"].reshape(1, -1), p["ln_g"].reshape(1, -1),
            p["ln_b"].reshape(1, -1))


def _node_w(p, lat):
    return (p["w0"][:lat], p["w0"][lat:], p["b0"].reshape(1, -1), p["w1"],
            p["b1"].reshape(1, -1), p["ln_g"].reshape(1, -1),
            p["ln_b"].reshape(1, -1))


def _gmp(x_list, dd, i_idx, j_idx, posk, pe, pn, n, lat, zz, extra=None):
    c3, c4, wa, wb, b0e, w1e, b1e, ge, be = _edge_w(pe, lat)
    a_t, b_t = _tc_prep(x_list, posk, wa, wb, c3)
    if dd is None:
        gab, dd = _sc_gather_combine(
            [(a_t, b_t, "add"), (posk, posk, "diff16")], i_idx, j_idx)
    else:
        (gab,) = _sc_gather_combine([(a_t, b_t, "add")], i_idx, j_idx)
    emb = _tc_edge(gab, dd, c4, b0e, w1e, b1e, ge, be)
    (agp,) = _sc_scatter([emb], j_idx, n, [zz])
    wx, wac, b0n, w1n, b1n, gn, bn = _node_w(pn, lat)
    out = _tc_node(x_list, agp[0], agp[1], wx, wac, b0n, w1n, b1n, gn, bn,
                   extra=extra)
    return out, dd


def kernel(h, m_ids, m_gs, pos, params):
    n, lat = h.shape
    e = m_gs.shape[-1]
    i0 = m_gs[0, 0]
    j0 = m_gs[0, 1]
    i1 = m_gs[1, 0]
    j1 = m_gs[1, 1]
    ids0 = m_ids[0]
    padn = ((n + NW * CH - 1) // (NW * CH)) * NW * CH

    posp = jnp.pad(pos, ((0, 0), (0, LAT - pos.shape[1])))
    zz128 = jnp.zeros((n, lat), F32)

    h1, d0 = _gmp([h], None, i0, j0, posp, params["down0"]["mlp_edge"],
                  params["down0"]["mlp_node"], n, lat, zz128)

    (degp,) = _sc_scatter(None, i0, n, [zz128], ones_width=lat)
    nw128 = _tc_combine(degp, [], post="inv")

    pt = _tc_postable(posp, nw128)
    sp = _sc_gather_scatter(pt, i0, j0, n, zz128)
    iaw128, p2f = _tc_unpack_s(sp)

    hw1 = _tc_mul(h1, nw128)
    shp = _sc_gather_scatter(hw1, i0, j0, n, zz128)
    h2f = _tc_combine(shp, [iaw128])
    idp = jnp.pad(ids0, (0, padn - n))
    h2p, p2p = _sc_gather([h2f, p2f], [idp, idp])
    h2 = h2p[:n]
    p2 = p2p[:n]

    h3, _ = _gmp([h2], None, i1, j1, p2, params["bottom"]["mlp_edge"],
                 params["bottom"]["mlp_node"], n, lat, zz128)

    winp = _sc_win(idp, n, padn)
    winc, maskf = _tc_winmax(winp)
    wpad = jnp.pad(winc.reshape(-1)[:n], (0, padn - n))
    (h4p,) = _sc_gather([h3], [wpad])

    hw4 = _tc_mul(h4p[:n], iaw128, col=maskf.reshape(-1)[:n].reshape(n, 1))
    up_ = _sc_gather_scatter(hw4, j0, i0, n, zz128)
    hu = _tc_combine(up_, [nw128])

    out, _ = _gmp([hu], d0, i0, j0, posp, params["up0"]["mlp_edge"],
                  params["up0"]["mlp_node"], n, lat, zz128, extra=h1)
    return out
